# Initial kernel scaffold; baseline (speedup 1.0000x reference)
import jax
import jax.numpy as jnp
from jax.experimental import pallas as pl


def kernel(x, attr, segment_edge, segment_node, params, s_index, e_index, edge_index, cur_t, cur_w):
    raise NotImplementedError("write your pallas kernel here")


# speedup vs baseline: 1.9329x; 1.9329x over previous
import functools
import math

import jax
import jax.numpy as jnp
import numpy as np
from jax.experimental import pallas as pl
from jax.experimental.pallas import tpu as pltpu

XMIN = -1.21
XMAX = 23.91
NUM_NODES = 10000
SQRT_2_PI = float(np.sqrt(2.0 / np.pi))


def _gelu(x):
    return 0.5 * x * (1.0 + jnp.tanh(SQRT_2_PI * (x + 0.044715 * x * x * x)))


def _act(r, act):
    if act == 'gelu':
        return _gelu(r)
    if act == 'leaky2':
        return jnp.where(r > 0, r, 0.2 * r)
    if act == 'leaky01':
        return jnp.where(r > 0, r, 0.01 * r)
    return r


def _rup(v, m):
    return ((v + m - 1) // m) * m


def _mm_body(a_ref, b_ref, bias_ref, o_ref, acc_ref, *, nk, act):
    k = pl.program_id(2)

    @pl.when(k == 0)
    def _():
        acc_ref[...] = jnp.zeros_like(acc_ref)

    acc_ref[...] += jnp.dot(a_ref[...], b_ref[...],
                            preferred_element_type=jnp.float32)

    @pl.when(k == nk - 1)
    def _():
        r = acc_ref[...] + bias_ref[...]
        o_ref[...] = _act(r, act)


@functools.partial(jax.jit, static_argnames=('act',))
def _mm(a, b, bias=None, act=None):
    M, K = a.shape
    K2, N = b.shape
    assert K == K2
    if bias is None:
        bias = jnp.zeros((N,), jnp.float32)
    bm = _rup(M, 8) if M < 256 else 256
    Mp = _rup(M, bm)
    Np = _rup(N, 128)
    bn = min(Np, 512)
    Np = _rup(Np, bn)
    if K <= 2048:
        bk = _rup(K, 8)
    else:
        bk = 3200 if K % 3200 == 0 else 2560
    Kp = _rup(K, bk)
    a = jnp.pad(a, ((0, Mp - M), (0, Kp - K)))
    b = jnp.pad(b, ((0, Kp - K), (0, Np - N)))
    bias = jnp.pad(bias, (0, Np - N)).reshape(1, Np)
    nk = Kp // bk
    out = pl.pallas_call(
        functools.partial(_mm_body, nk=nk, act=act),
        grid=(Mp // bm, Np // bn, nk),
        in_specs=[
            pl.BlockSpec((bm, bk), lambda m, n, k: (m, k)),
            pl.BlockSpec((bk, bn), lambda m, n, k: (k, n)),
            pl.BlockSpec((1, bn), lambda m, n, k: (0, n)),
        ],
        out_specs=pl.BlockSpec((bm, bn), lambda m, n, k: (m, n)),
        out_shape=jax.ShapeDtypeStruct((Mp, Np), jnp.float32),
        scratch_shapes=[pltpu.VMEM((bm, bn), jnp.float32)],
        compiler_params=pltpu.CompilerParams(
            dimension_semantics=("parallel", "parallel", "arbitrary")),
    )(a, b, bias)
    return out[:M, :N]


def _edge_ex_body(xls_ref, xrd_ref, pe_ref, att_ref, ex_ref, y_ref):
    s = xls_ref[...] + xrd_ref[...] + pe_ref[...]
    m = jnp.where(s > 0, s, 0.2 * s)
    ex = jnp.exp(jnp.sum(m * att_ref[...], axis=1, keepdims=True))
    ex_ref[...] = ex
    y_ref[...] = ex * xls_ref[...]


@jax.jit
def _edge_ex(xlsrc, xrdst, pe, att):
    E, H = xlsrc.shape
    bm = 1000
    ex, y = pl.pallas_call(
        _edge_ex_body,
        grid=(E // bm,),
        in_specs=[
            pl.BlockSpec((bm, H), lambda m: (m, 0)),
            pl.BlockSpec((bm, H), lambda m: (m, 0)),
            pl.BlockSpec((bm, H), lambda m: (m, 0)),
            pl.BlockSpec((1, H), lambda m: (0, 0)),
        ],
        out_specs=[
            pl.BlockSpec((bm, 1), lambda m: (m, 0)),
            pl.BlockSpec((bm, H), lambda m: (m, 0)),
        ],
        out_shape=[
            jax.ShapeDtypeStruct((E, 1), jnp.float32),
            jax.ShapeDtypeStruct((E, H), jnp.float32),
        ],
        compiler_params=pltpu.CompilerParams(
            dimension_semantics=("parallel",)),
    )(xlsrc, xrdst, pe, att.reshape(1, H))
    return ex[:, 0], y


def _combine_body(xl_ref, xr_ref, pel_ref, att_ref, num_ref, den_ref,
                  bias_ref, pre_ref, o_ref):
    xl = xl_ref[...]
    s = xl + xr_ref[...] + pel_ref[...]
    m = jnp.where(s > 0, s, 0.2 * s)
    exl = jnp.exp(jnp.sum(m * att_ref[...], axis=1, keepdims=True))
    r = (num_ref[...] + exl * xl) / (den_ref[...] + exl + 1e-16)
    r = r + bias_ref[...]
    o_ref[...] = _gelu(r) + pre_ref[...]


@jax.jit
def _gat_combine(xl, xr, pe_loop, att, num, den, bias, pre):
    N, H = xl.shape
    bm = 1000
    return pl.pallas_call(
        _combine_body,
        grid=(N // bm,),
        in_specs=[
            pl.BlockSpec((bm, H), lambda m: (m, 0)),
            pl.BlockSpec((bm, H), lambda m: (m, 0)),
            pl.BlockSpec((bm, H), lambda m: (m, 0)),
            pl.BlockSpec((1, H), lambda m: (0, 0)),
            pl.BlockSpec((bm, H), lambda m: (m, 0)),
            pl.BlockSpec((bm, 1), lambda m: (m, 0)),
            pl.BlockSpec((1, H), lambda m: (0, 0)),
            pl.BlockSpec((bm, H), lambda m: (m, 0)),
        ],
        out_specs=pl.BlockSpec((bm, H), lambda m: (m, 0)),
        out_shape=jax.ShapeDtypeStruct((N, H), jnp.float32),
        compiler_params=pltpu.CompilerParams(
            dimension_semantics=("parallel",)),
    )(xl, xr, pe_loop, att.reshape(1, H), num, den.reshape(N, 1),
      bias.reshape(1, H), pre)


def _gat_big(xin, src, dst, attr, am16, p, lin_W, lin_b, pre):
    Wc = _mm(lin_W, p['We'])
    bc = lin_b @ p['We']
    xl = _mm(xin, p['Wl'], p['bl'])
    xr = _mm(xin, p['Wr'], p['br'])
    pe = _mm(attr, Wc, bc)
    pe_loop = _mm(am16, Wc, bc)
    xlsrc = jnp.take(xl, src, axis=0)
    xrdst = jnp.take(xr, dst, axis=0)
    ex, y = _edge_ex(xlsrc, xrdst, pe, p['att'])
    den = jax.ops.segment_sum(ex, dst, num_segments=NUM_NODES)
    num = jax.ops.segment_sum(y, dst, num_segments=NUM_NODES)
    return _gat_combine(xl, xr, pe_loop, p['att'], num, den, p['bias'], pre)


def _gat_seg(xf, mm, p):
    N = xf.shape[0]
    mask = mm != 0.0
    cnt = jnp.sum(mask.astype(jnp.float32), axis=0)
    loop_attr = jnp.sum(mm, axis=0) / jnp.maximum(cnt, 1.0)
    xl = _mm(xf, p['Wl'], p['bl'])
    xr = _mm(xf, p['Wr'], p['br'])
    We = p['We'][0]
    s = xl[:, None, :] + xr[None, :, :] + mm[:, :, None] * We[None, None, :]
    m = jnp.where(s > 0, s, 0.2 * s)
    alpha = jnp.einsum('ijh,h->ij', m, p['att'])
    alpha = jnp.where(mask, alpha, -jnp.inf)
    sl = xl + xr + loop_attr[:, None] * We[None, :]
    ml = jnp.where(sl > 0, sl, 0.2 * sl)
    aloop = ml @ p['att']
    amax = jnp.maximum(jnp.max(jnp.where(mask, alpha, -jnp.inf), axis=0), aloop)
    ex = jnp.where(mask, jnp.exp(alpha - amax[None, :]), 0.0)
    exl = jnp.exp(aloop - amax)
    den = jnp.sum(ex, axis=0) + exl + 1e-16
    out = (ex.T @ xl + exl[:, None] * xl) / den[:, None]
    return out + p['bias']


def kernel(x, attr, segment_edge, segment_node, params, s_index, e_index,
           edge_index, cur_t, cur_w):
    p = params
    mm = segment_node @ segment_node.T
    src, dst = edge_index[0], edge_index[1]

    mask_idx = (jnp.sum(x, axis=1, keepdims=True) != XMIN * 4).astype(jnp.float32)
    ratio = 0.8 + 0.4 * jax.random.uniform(jax.random.key(1))
    drop_idx = (jax.random.uniform(jax.random.key(2), (x.shape[0], 1)) > 0.3
                ).astype(jnp.float32)
    x_norm = ((x - XMIN) / (XMAX - XMIN) * ratio * drop_idx).T
    h = _mm(x_norm, p['fc1_W'], p['fc1_b'], act='gelu')
    mu = _mm(h, p['fc2_W'], p['fc2_b'])
    log_var = _mm(h, p['fc3_W'], p['fc3_b'])
    eps = jax.random.normal(jax.random.key(3), mu.shape, jnp.float32)
    z = mu + eps * jnp.exp(log_var / 2)
    hz = _mm(z, p['fc4_W'], p['fc4_b'], act='gelu')
    x_rec = _mm(hz, p['fc5_W'], p['fc5_b'])
    x_rec = (x_rec / ratio).T
    x_rec = x_rec * (XMAX - XMIN) + XMIN
    x_rec1 = mask_idx * x + (1 - mask_idx) * x_rec

    ones = jnp.ones(dst.shape[0], jnp.float32)
    cnt = jax.ops.segment_sum(ones, dst, num_segments=NUM_NODES)
    am16 = (jax.ops.segment_sum(attr, dst, num_segments=NUM_NODES)
            / jnp.maximum(cnt, 1.0)[:, None])

    ne = p['node_embed']
    pre = ne
    for name in ('conv1_0', 'conv1_1'):
        ne = _gat_big(ne, src, dst, attr, am16, p[name],
                      p['attr_lin_W'], p['attr_lin_b'], pre)
    data = _mm(x_rec1, p['node_lin_W'], p['node_lin_b'], act='gelu')
    pre = data
    for name in ('conv2_0', 'conv2_1'):
        data = _gat_big(data, src, dst, attr, am16, p[name],
                        p['attr_lin_W'], p['attr_lin_b'], pre)

    time_embed = p['time_embed'][cur_t]
    week_embed = p['week_embed'][cur_w]
    seg_embed = p['segment_embed'][s_index]
    h1 = _mm(attr, p['attr1_W1'], p['attr1_b1'], act='leaky01')
    attr2 = (_mm(_mm(segment_edge, h1), p['attr1_W2'])
             + jnp.sum(segment_edge, axis=1, keepdims=True)
             * p['attr1_b2'][None, :])
    embed = jnp.take(p['edge_embed'], e_index, axis=0)
    attr3 = _mm(segment_edge, embed)
    xnh = _mm(segment_node, jnp.concatenate([data, ne], axis=1))
    x2, x1 = xnh[:, :128], xnh[:, 128:]
    xf = jnp.concatenate([seg_embed, time_embed, attr2, attr3, x2, x1,
                          week_embed], axis=1)
    xf1 = _gelu(_gat_seg(xf, mm, p['conv3']))
    for name in ('lin0', 'lin1', 'lin2'):
        xf = _mm(xf, p[name + '_W'], p[name + '_b'], act='gelu')
    xf = _mm(jnp.concatenate([xf, xf1], axis=1), p['lin3_W'], p['lin3_b'])
    out = jax.nn.sigmoid(xf) * 3600.0
    return out, x_rec


# speedup vs baseline: 2.7306x; 1.4127x over previous
import functools
import math

import jax
import jax.numpy as jnp
import numpy as np
from jax import lax
from jax.experimental import pallas as pl
from jax.experimental.pallas import tpu as pltpu
from jax.experimental.pallas import tpu_sc as plsc

XMIN = -1.21
XMAX = 23.91
NUM_NODES = 10000
SQRT_2_PI = float(np.sqrt(2.0 / np.pi))

_NC, _NS = 2, 16
_NW = _NC * _NS
_E = 160000
_EP = 163840
_PW = _EP // _NW
_CH = 128
_NCH = _PW // _CH
_NBIN = 10112
_RPS = _NBIN // _NS
_HE = 128
_SC_MESH = plsc.VectorSubcoreMesh(core_axis_name="c", subcore_axis_name="s")


@functools.partial(
    pl.kernel,
    out_type=[jax.ShapeDtypeStruct((_EP, 128), jnp.float32),
              jax.ShapeDtypeStruct((_EP, 128), jnp.float32)],
    mesh=_SC_MESH,
    scratch_types=[pltpu.VMEM((_CH,), jnp.int32),
                   pltpu.VMEM((_CH, 128), jnp.float32),
                   pltpu.SemaphoreType.DMA],
)
def _sc_gather2(xl_hbm, xr_hbm, src_hbm, dst_hbm, o1_hbm, o2_hbm,
                idx_v, rows_v, sem):
    wid = lax.axis_index("s") * _NC + lax.axis_index("c")
    base = wid * _PW

    def body(i, carry):
        off = base + i * _CH
        pltpu.sync_copy(src_hbm.at[pl.ds(off, _CH)], idx_v)
        pltpu.async_copy(xl_hbm.at[idx_v], rows_v, sem).wait()
        pltpu.sync_copy(rows_v, o1_hbm.at[pl.ds(off, _CH)])
        pltpu.sync_copy(dst_hbm.at[pl.ds(off, _CH)], idx_v)
        pltpu.async_copy(xr_hbm.at[idx_v], rows_v, sem).wait()
        pltpu.sync_copy(rows_v, o2_hbm.at[pl.ds(off, _CH)])
        return carry

    lax.fori_loop(0, _NCH, body, 0)


@functools.partial(
    pl.kernel,
    out_type=jax.ShapeDtypeStruct((_NC, _NBIN, _HE), jnp.float32),
    mesh=_SC_MESH,
    scratch_types=[pltpu.VMEM((_CH,), jnp.int32),
                   pltpu.VMEM((_CH, _HE), jnp.float32),
                   pltpu.VMEM_SHARED((_NBIN, _HE), jnp.float32),
                   pltpu.SemaphoreType.DMA],
)
def _sc_scatter(y_hbm, dst_hbm, z_hbm, out_hbm, idx_v, rows_v, acc_sh, sem):
    cid = lax.axis_index("c")
    sid = lax.axis_index("s")
    wid = sid * _NC + cid
    r0 = sid * _RPS
    pltpu.sync_copy(z_hbm.at[pl.ds(r0, _RPS)], acc_sh.at[pl.ds(r0, _RPS)])
    plsc.subcore_barrier()
    base = wid * _PW

    def body(i, carry):
        off = base + i * _CH
        pltpu.sync_copy(dst_hbm.at[pl.ds(off, _CH)], idx_v)
        pltpu.sync_copy(y_hbm.at[pl.ds(off, _CH)], rows_v)
        pltpu.sync_copy(rows_v, acc_sh.at[idx_v], add=True)
        return carry

    lax.fori_loop(0, _NCH, body, 0)
    plsc.subcore_barrier()
    pltpu.sync_copy(acc_sh.at[pl.ds(r0, _RPS)],
                    out_hbm.at[cid].at[pl.ds(r0, _RPS)])


def _gelu(x):
    return 0.5 * x * (1.0 + jnp.tanh(SQRT_2_PI * (x + 0.044715 * x * x * x)))


def _act(r, act):
    if act == 'gelu':
        return _gelu(r)
    if act == 'leaky2':
        return jnp.where(r > 0, r, 0.2 * r)
    if act == 'leaky01':
        return jnp.where(r > 0, r, 0.01 * r)
    return r


def _rup(v, m):
    return ((v + m - 1) // m) * m


def _mm_body(a_ref, b_ref, bias_ref, o_ref, acc_ref, *, nk, act):
    k = pl.program_id(2)

    @pl.when(k == 0)
    def _():
        acc_ref[...] = jnp.zeros_like(acc_ref)

    acc_ref[...] += jnp.dot(a_ref[...], b_ref[...],
                            preferred_element_type=jnp.float32)

    @pl.when(k == nk - 1)
    def _():
        r = acc_ref[...] + bias_ref[...]
        o_ref[...] = _act(r, act)


@functools.partial(jax.jit, static_argnames=('act',))
def _mm(a, b, bias=None, act=None):
    M, K = a.shape
    K2, N = b.shape
    assert K == K2
    if bias is None:
        bias = jnp.zeros((N,), jnp.float32)
    bm = _rup(M, 8) if M < 256 else 256
    Mp = _rup(M, bm)
    Np = _rup(N, 128)
    bn = min(Np, 512)
    Np = _rup(Np, bn)
    if K <= 2048:
        bk = _rup(K, 8)
    else:
        bk = 3200 if K % 3200 == 0 else 2560
    Kp = _rup(K, bk)
    a = jnp.pad(a, ((0, Mp - M), (0, Kp - K)))
    b = jnp.pad(b, ((0, Kp - K), (0, Np - N)))
    bias = jnp.pad(bias, (0, Np - N)).reshape(1, Np)
    nk = Kp // bk
    out = pl.pallas_call(
        functools.partial(_mm_body, nk=nk, act=act),
        grid=(Mp // bm, Np // bn, nk),
        in_specs=[
            pl.BlockSpec((bm, bk), lambda m, n, k: (m, k)),
            pl.BlockSpec((bk, bn), lambda m, n, k: (k, n)),
            pl.BlockSpec((1, bn), lambda m, n, k: (0, n)),
        ],
        out_specs=pl.BlockSpec((bm, bn), lambda m, n, k: (m, n)),
        out_shape=jax.ShapeDtypeStruct((Mp, Np), jnp.float32),
        scratch_shapes=[pltpu.VMEM((bm, bn), jnp.float32)],
        compiler_params=pltpu.CompilerParams(
            dimension_semantics=("parallel", "parallel", "arbitrary")),
    )(a, b, bias)
    return out[:M, :N]


def _edge_y_body(xls_ref, xrd_ref, pe_ref, att_ref, y_ref, ex_ref, *, bm):
    xls = xls_ref[...]
    s = xls + xrd_ref[...] + pe_ref[...]
    m = jnp.where(s > 0, s, 0.2 * s)
    ex = jnp.exp(jnp.sum(m * att_ref[...], axis=1, keepdims=True))
    row = (pl.program_id(0) * bm
           + lax.broadcasted_iota(jnp.int32, (bm, 1), 0))
    ex = jnp.where(row < _E, ex, 0.0)
    y_ref[...] = ex * xls
    ex_ref[...] = ex


@jax.jit
def _edge_y(xlsrc, xrdst, pe, att):
    E, H = xlsrc.shape
    bm = 1280
    return pl.pallas_call(
        functools.partial(_edge_y_body, bm=bm),
        grid=(E // bm,),
        in_specs=[
            pl.BlockSpec((bm, H), lambda m: (m, 0)),
            pl.BlockSpec((bm, H), lambda m: (m, 0)),
            pl.BlockSpec((bm, H), lambda m: (m, 0)),
            pl.BlockSpec((1, H), lambda m: (0, 0)),
        ],
        out_specs=[
            pl.BlockSpec((bm, _HE), lambda m: (m, 0)),
            pl.BlockSpec((bm, 1), lambda m: (m, 0)),
        ],
        out_shape=[
            jax.ShapeDtypeStruct((E, _HE), jnp.float32),
            jax.ShapeDtypeStruct((E, 1), jnp.float32),
        ],
        compiler_params=pltpu.CompilerParams(
            dimension_semantics=("parallel",)),
    )(xlsrc, xrdst, pe, att.reshape(1, H))


def _combine_body(xl_ref, xr_ref, pel_ref, att_ref, p0_ref, p1_ref,
                  den_ref, bias_ref, pre_ref, o_ref):
    xl = xl_ref[...]
    s = xl + xr_ref[...] + pel_ref[...]
    m = jnp.where(s > 0, s, 0.2 * s)
    exl = jnp.exp(jnp.sum(m * att_ref[...], axis=1, keepdims=True))
    num = p0_ref[0] + p1_ref[0]
    r = (num + exl * xl) / (den_ref[...] + exl + 1e-16)
    r = r + bias_ref[...]
    o_ref[...] = _gelu(r) + pre_ref[...]


@jax.jit
def _gat_combine(xl, xr, pe_loop, att, parts, den, bias, pre):
    N, H = xl.shape
    bm = 1000
    return pl.pallas_call(
        _combine_body,
        grid=(N // bm,),
        in_specs=[
            pl.BlockSpec((bm, H), lambda m: (m, 0)),
            pl.BlockSpec((bm, H), lambda m: (m, 0)),
            pl.BlockSpec((bm, H), lambda m: (m, 0)),
            pl.BlockSpec((1, H), lambda m: (0, 0)),
            pl.BlockSpec((1, bm, _HE), lambda m: (0, m, 0)),
            pl.BlockSpec((1, bm, _HE), lambda m: (1, m, 0)),
            pl.BlockSpec((bm, 1), lambda m: (m, 0)),
            pl.BlockSpec((1, H), lambda m: (0, 0)),
            pl.BlockSpec((bm, H), lambda m: (m, 0)),
        ],
        out_specs=pl.BlockSpec((bm, H), lambda m: (m, 0)),
        out_shape=jax.ShapeDtypeStruct((N, H), jnp.float32),
        compiler_params=pltpu.CompilerParams(
            dimension_semantics=("parallel",)),
    )(xl, xr, pe_loop, att.reshape(1, H), parts, parts,
      den.reshape(N, 1), bias.reshape(1, H), pre)


def _gat_big(xin, src_pad, dst_pad, dst_bin, attr_pad, am16, p,
             lin_W, lin_b, pre, zeros_acc):
    Wc = _mm(lin_W, p['We'])
    bc = lin_b @ p['We']
    xl = _mm(xin, p['Wl'], p['bl'])
    xr = _mm(xin, p['Wr'], p['br'])
    pe = _mm(attr_pad, Wc, bc)
    pe_loop = _mm(am16, Wc, bc)
    xlsrc, xrdst = _sc_gather2(xl, xr, src_pad, dst_pad)
    y, ex = _edge_y(xlsrc, xrdst, pe, p['att'])
    parts = _sc_scatter(y, dst_bin, zeros_acc)
    den = jax.ops.segment_sum(ex[:_E, 0], dst_pad[:_E],
                              num_segments=NUM_NODES)
    return _gat_combine(xl, xr, pe_loop, p['att'], parts, den,
                        p['bias'], pre)


def _gat_seg(xf, mm, p):
    N = xf.shape[0]
    mask = mm != 0.0
    cnt = jnp.sum(mask.astype(jnp.float32), axis=0)
    loop_attr = jnp.sum(mm, axis=0) / jnp.maximum(cnt, 1.0)
    xl = _mm(xf, p['Wl'], p['bl'])
    xr = _mm(xf, p['Wr'], p['br'])
    We = p['We'][0]
    s = xl[:, None, :] + xr[None, :, :] + mm[:, :, None] * We[None, None, :]
    m = jnp.where(s > 0, s, 0.2 * s)
    alpha = jnp.einsum('ijh,h->ij', m, p['att'])
    alpha = jnp.where(mask, alpha, -jnp.inf)
    sl = xl + xr + loop_attr[:, None] * We[None, :]
    ml = jnp.where(sl > 0, sl, 0.2 * sl)
    aloop = ml @ p['att']
    amax = jnp.maximum(jnp.max(jnp.where(mask, alpha, -jnp.inf), axis=0), aloop)
    ex = jnp.where(mask, jnp.exp(alpha - amax[None, :]), 0.0)
    exl = jnp.exp(aloop - amax)
    den = jnp.sum(ex, axis=0) + exl + 1e-16
    out = (ex.T @ xl + exl[:, None] * xl) / den[:, None]
    return out + p['bias']


def kernel(x, attr, segment_edge, segment_node, params, s_index, e_index,
           edge_index, cur_t, cur_w):
    p = params
    mm = segment_node @ segment_node.T
    src, dst = edge_index[0], edge_index[1]

    mask_idx = (jnp.sum(x, axis=1, keepdims=True) != XMIN * 4).astype(jnp.float32)
    ratio = 0.8 + 0.4 * jax.random.uniform(jax.random.key(1))
    drop_idx = (jax.random.uniform(jax.random.key(2), (x.shape[0], 1)) > 0.3
                ).astype(jnp.float32)
    x_norm = ((x - XMIN) / (XMAX - XMIN) * ratio * drop_idx).T
    h = _mm(x_norm, p['fc1_W'], p['fc1_b'], act='gelu')
    mu = _mm(h, p['fc2_W'], p['fc2_b'])
    log_var = _mm(h, p['fc3_W'], p['fc3_b'])
    eps = jax.random.normal(jax.random.key(3), mu.shape, jnp.float32)
    z = mu + eps * jnp.exp(log_var / 2)
    hz = _mm(z, p['fc4_W'], p['fc4_b'], act='gelu')
    x_rec = _mm(hz, p['fc5_W'], p['fc5_b'])
    x_rec = (x_rec / ratio).T
    x_rec = x_rec * (XMAX - XMIN) + XMIN
    x_rec1 = mask_idx * x + (1 - mask_idx) * x_rec

    ones = jnp.ones(dst.shape[0], jnp.float32)
    cnt = jax.ops.segment_sum(ones, dst, num_segments=NUM_NODES)
    am16 = (jax.ops.segment_sum(attr, dst, num_segments=NUM_NODES)
            / jnp.maximum(cnt, 1.0)[:, None])

    npad = _EP - _E
    src_pad = jnp.pad(src, (0, npad)).astype(jnp.int32)
    dst_pad = jnp.pad(dst, (0, npad)).astype(jnp.int32)
    dst_bin = jnp.pad(dst, (0, npad),
                      constant_values=NUM_NODES).astype(jnp.int32)
    attr_pad = jnp.pad(attr, ((0, npad), (0, 0)))
    zeros_acc = jnp.zeros((_NBIN, _HE), jnp.float32)

    ne = p['node_embed']
    pre = ne
    for name in ('conv1_0', 'conv1_1'):
        ne = _gat_big(ne, src_pad, dst_pad, dst_bin, attr_pad, am16, p[name],
                      p['attr_lin_W'], p['attr_lin_b'], pre, zeros_acc)
    data = _mm(x_rec1, p['node_lin_W'], p['node_lin_b'], act='gelu')
    pre = data
    for name in ('conv2_0', 'conv2_1'):
        data = _gat_big(data, src_pad, dst_pad, dst_bin, attr_pad, am16,
                        p[name], p['attr_lin_W'], p['attr_lin_b'], pre,
                        zeros_acc)

    time_embed = p['time_embed'][cur_t]
    week_embed = p['week_embed'][cur_w]
    seg_embed = p['segment_embed'][s_index]
    h1 = _mm(attr, p['attr1_W1'], p['attr1_b1'], act='leaky01')
    attr2 = (_mm(_mm(segment_edge, h1), p['attr1_W2'])
             + jnp.sum(segment_edge, axis=1, keepdims=True)
             * p['attr1_b2'][None, :])
    embed = jnp.take(p['edge_embed'], e_index, axis=0)
    attr3 = _mm(segment_edge, embed)
    xnh = _mm(segment_node, jnp.concatenate([data, ne], axis=1))
    x2, x1 = xnh[:, :128], xnh[:, 128:]
    xf = jnp.concatenate([seg_embed, time_embed, attr2, attr3, x2, x1,
                          week_embed], axis=1)
    xf1 = _gelu(_gat_seg(xf, mm, p['conv3']))
    for name in ('lin0', 'lin1', 'lin2'):
        xf = _mm(xf, p[name + '_W'], p[name + '_b'], act='gelu')
    xf = _mm(jnp.concatenate([xf, xf1], axis=1), p['lin3_W'], p['lin3_b'])
    out = jax.nn.sigmoid(xf) * 3600.0
    return out, x_rec


# speedup vs baseline: 2.9400x; 1.0767x over previous
import functools
import math

import jax
import jax.numpy as jnp
import numpy as np
from jax import lax
from jax.experimental import pallas as pl
from jax.experimental.pallas import tpu as pltpu
from jax.experimental.pallas import tpu_sc as plsc

XMIN = -1.21
XMAX = 23.91
NUM_NODES = 10000
SQRT_2_PI = float(np.sqrt(2.0 / np.pi))

_NC, _NS = 2, 16
_NW = _NC * _NS
_E = 160000
_EP = 163840
_PW = _EP // _NW
_CH = 128
_NCH = _PW // _CH
_NBIN = 10112
_RPS = _NBIN // _NS
_HE = 128
_SC_MESH = plsc.VectorSubcoreMesh(core_axis_name="c", subcore_axis_name="s")


_GNB = 4
_GIT = 2 * _NCH // _GNB


@functools.partial(
    pl.kernel,
    out_type=[jax.ShapeDtypeStruct((_EP, 128), jnp.float32),
              jax.ShapeDtypeStruct((_EP, 128), jnp.float32)],
    mesh=_SC_MESH,
    scratch_types=([pltpu.VMEM((_CH,), jnp.int32)] * _GNB
                   + [pltpu.VMEM((_CH, 128), jnp.float32)] * _GNB
                   + [pltpu.SemaphoreType.DMA] * _GNB),
)
def _sc_gather2(xl_hbm, xr_hbm, src_hbm, dst_hbm, o1_hbm, o2_hbm, *scr):
    idxs = scr[:_GNB]
    rows = scr[_GNB:2 * _GNB]
    sems = scr[2 * _GNB:]
    wid = lax.axis_index("s") * _NC + lax.axis_index("c")
    base = wid * _PW
    tabs = [xl_hbm, xr_hbm]
    srcs = [src_hbm, dst_hbm]
    outs = [o1_hbm, o2_hbm]

    for b in range(_GNB):
        off = base + (b // 2) * _CH
        pltpu.sync_copy(srcs[b % 2].at[pl.ds(off, _CH)], idxs[b])
        pltpu.async_copy(tabs[b % 2].at[idxs[b]], rows[b], sems[b])

    def body(k, carry):
        for b in range(_GNB):
            off = base + (b // 2 + (_GNB // 2) * k) * _CH
            pltpu.make_async_copy(tabs[b % 2].at[idxs[b]], rows[b],
                                  sems[b]).wait()
            pltpu.sync_copy(rows[b], outs[b % 2].at[pl.ds(off, _CH)])

            @pl.when(k + 1 < _GIT)
            def _():
                off2 = off + (_GNB // 2) * _CH
                pltpu.sync_copy(srcs[b % 2].at[pl.ds(off2, _CH)], idxs[b])
                pltpu.async_copy(tabs[b % 2].at[idxs[b]], rows[b], sems[b])
        return carry

    lax.fori_loop(0, _GIT, body, 0)


_SNB = 2


@functools.partial(
    pl.kernel,
    out_type=jax.ShapeDtypeStruct((_NC, _NBIN, _HE), jnp.float32),
    mesh=_SC_MESH,
    scratch_types=([pltpu.VMEM((_CH,), jnp.int32)] * _SNB
                   + [pltpu.VMEM((_CH, _HE), jnp.float32)] * _SNB
                   + [pltpu.SemaphoreType.DMA] * _SNB
                   + [pltpu.VMEM_SHARED((_NBIN, _HE), jnp.float32)]),
)
def _sc_scatter(y_hbm, dst_hbm, z_hbm, out_hbm, *scr):
    idxs = scr[:_SNB]
    rows = scr[_SNB:2 * _SNB]
    sems = scr[2 * _SNB:3 * _SNB]
    acc_sh = scr[3 * _SNB]
    cid = lax.axis_index("c")
    sid = lax.axis_index("s")
    wid = sid * _NC + cid
    r0 = sid * _RPS
    base = wid * _PW

    for b in range(_SNB):
        off = base + b * _CH
        pltpu.sync_copy(dst_hbm.at[pl.ds(off, _CH)], idxs[b])
        pltpu.async_copy(y_hbm.at[pl.ds(off, _CH)], rows[b], sems[b])

    pltpu.sync_copy(z_hbm.at[pl.ds(r0, _RPS)], acc_sh.at[pl.ds(r0, _RPS)])
    plsc.subcore_barrier()

    def body(k, carry):
        for b in range(_SNB):
            off = base + (b + _SNB * k) * _CH
            pltpu.make_async_copy(y_hbm.at[pl.ds(off, _CH)], rows[b],
                                  sems[b]).wait()
            pltpu.sync_copy(rows[b], acc_sh.at[idxs[b]], add=True)

            @pl.when(k + 1 < _NCH // _SNB)
            def _():
                off2 = off + _SNB * _CH
                pltpu.sync_copy(dst_hbm.at[pl.ds(off2, _CH)], idxs[b])
                pltpu.async_copy(y_hbm.at[pl.ds(off2, _CH)], rows[b], sems[b])
        return carry

    lax.fori_loop(0, _NCH // _SNB, body, 0)
    plsc.subcore_barrier()
    pltpu.sync_copy(acc_sh.at[pl.ds(r0, _RPS)],
                    out_hbm.at[cid].at[pl.ds(r0, _RPS)])


def _gelu(x):
    return 0.5 * x * (1.0 + jnp.tanh(SQRT_2_PI * (x + 0.044715 * x * x * x)))


def _act(r, act):
    if act == 'gelu':
        return _gelu(r)
    if act == 'leaky2':
        return jnp.where(r > 0, r, 0.2 * r)
    if act == 'leaky01':
        return jnp.where(r > 0, r, 0.01 * r)
    return r


def _rup(v, m):
    return ((v + m - 1) // m) * m


def _mm_body(a_ref, b_ref, bias_ref, o_ref, acc_ref, *, nk, act):
    k = pl.program_id(2)

    @pl.when(k == 0)
    def _():
        acc_ref[...] = jnp.zeros_like(acc_ref)

    acc_ref[...] += jnp.dot(a_ref[...], b_ref[...],
                            preferred_element_type=jnp.float32)

    @pl.when(k == nk - 1)
    def _():
        r = acc_ref[...] + bias_ref[...]
        o_ref[...] = _act(r, act)


@functools.partial(jax.jit, static_argnames=('act',))
def _mm(a, b, bias=None, act=None):
    M, K = a.shape
    K2, N = b.shape
    assert K == K2
    if bias is None:
        bias = jnp.zeros((N,), jnp.float32)
    bm = _rup(M, 8) if M < 256 else 256
    Mp = _rup(M, bm)
    Np = _rup(N, 128)
    bn = min(Np, 512)
    Np = _rup(Np, bn)
    if K <= 2048:
        bk = _rup(K, 8)
    else:
        bk = 3200 if K % 3200 == 0 else 2560
    Kp = _rup(K, bk)
    a = jnp.pad(a, ((0, Mp - M), (0, Kp - K)))
    b = jnp.pad(b, ((0, Kp - K), (0, Np - N)))
    bias = jnp.pad(bias, (0, Np - N)).reshape(1, Np)
    nk = Kp // bk
    out = pl.pallas_call(
        functools.partial(_mm_body, nk=nk, act=act),
        grid=(Mp // bm, Np // bn, nk),
        in_specs=[
            pl.BlockSpec((bm, bk), lambda m, n, k: (m, k)),
            pl.BlockSpec((bk, bn), lambda m, n, k: (k, n)),
            pl.BlockSpec((1, bn), lambda m, n, k: (0, n)),
        ],
        out_specs=pl.BlockSpec((bm, bn), lambda m, n, k: (m, n)),
        out_shape=jax.ShapeDtypeStruct((Mp, Np), jnp.float32),
        scratch_shapes=[pltpu.VMEM((bm, bn), jnp.float32)],
        compiler_params=pltpu.CompilerParams(
            dimension_semantics=("parallel", "parallel", "arbitrary")),
    )(a, b, bias)
    return out[:M, :N]


def _edge_y_body(xls_ref, xrd_ref, pe_ref, att_ref, y_ref, ex_ref, *, bm):
    xls = xls_ref[...]
    s = xls + xrd_ref[...] + pe_ref[...]
    m = jnp.where(s > 0, s, 0.2 * s)
    ex = jnp.exp(jnp.sum(m * att_ref[...], axis=1, keepdims=True))
    row = (pl.program_id(0) * bm
           + lax.broadcasted_iota(jnp.int32, (bm, 1), 0))
    ex = jnp.where(row < _E, ex, 0.0)
    y_ref[...] = ex * xls
    ex_ref[...] = ex


@jax.jit
def _edge_y(xlsrc, xrdst, pe, att):
    E, H = xlsrc.shape
    bm = 1280
    return pl.pallas_call(
        functools.partial(_edge_y_body, bm=bm),
        grid=(E // bm,),
        in_specs=[
            pl.BlockSpec((bm, H), lambda m: (m, 0)),
            pl.BlockSpec((bm, H), lambda m: (m, 0)),
            pl.BlockSpec((bm, H), lambda m: (m, 0)),
            pl.BlockSpec((1, H), lambda m: (0, 0)),
        ],
        out_specs=[
            pl.BlockSpec((bm, _HE), lambda m: (m, 0)),
            pl.BlockSpec((bm, 1), lambda m: (m, 0)),
        ],
        out_shape=[
            jax.ShapeDtypeStruct((E, _HE), jnp.float32),
            jax.ShapeDtypeStruct((E, 1), jnp.float32),
        ],
        compiler_params=pltpu.CompilerParams(
            dimension_semantics=("parallel",)),
    )(xlsrc, xrdst, pe, att.reshape(1, H))


def _combine_body(xl_ref, xr_ref, pel_ref, att_ref, p0_ref, p1_ref,
                  den_ref, bias_ref, pre_ref, o_ref):
    xl = xl_ref[...]
    s = xl + xr_ref[...] + pel_ref[...]
    m = jnp.where(s > 0, s, 0.2 * s)
    exl = jnp.exp(jnp.sum(m * att_ref[...], axis=1, keepdims=True))
    num = p0_ref[0] + p1_ref[0]
    r = (num + exl * xl) / (den_ref[...] + exl + 1e-16)
    r = r + bias_ref[...]
    o_ref[...] = _gelu(r) + pre_ref[...]


@jax.jit
def _gat_combine(xl, xr, pe_loop, att, parts, den, bias, pre):
    N, H = xl.shape
    bm = 1000
    return pl.pallas_call(
        _combine_body,
        grid=(N // bm,),
        in_specs=[
            pl.BlockSpec((bm, H), lambda m: (m, 0)),
            pl.BlockSpec((bm, H), lambda m: (m, 0)),
            pl.BlockSpec((bm, H), lambda m: (m, 0)),
            pl.BlockSpec((1, H), lambda m: (0, 0)),
            pl.BlockSpec((1, bm, _HE), lambda m: (0, m, 0)),
            pl.BlockSpec((1, bm, _HE), lambda m: (1, m, 0)),
            pl.BlockSpec((bm, 1), lambda m: (m, 0)),
            pl.BlockSpec((1, H), lambda m: (0, 0)),
            pl.BlockSpec((bm, H), lambda m: (m, 0)),
        ],
        out_specs=pl.BlockSpec((bm, H), lambda m: (m, 0)),
        out_shape=jax.ShapeDtypeStruct((N, H), jnp.float32),
        compiler_params=pltpu.CompilerParams(
            dimension_semantics=("parallel",)),
    )(xl, xr, pe_loop, att.reshape(1, H), parts, parts,
      den.reshape(N, 1), bias.reshape(1, H), pre)


def _gat_big(xin, src_pad, dst_pad, dst_bin, attr_pad, am16, p,
             lin_W, lin_b, pre, zeros_acc):
    Wc = _mm(lin_W, p['We'])
    bc = lin_b @ p['We']
    xl = _mm(xin, p['Wl'], p['bl'])
    xr = _mm(xin, p['Wr'], p['br'])
    pe = _mm(attr_pad, Wc, bc)
    pe_loop = _mm(am16, Wc, bc)
    xlsrc, xrdst = _sc_gather2(xl, xr, src_pad, dst_pad)
    y, ex = _edge_y(xlsrc, xrdst, pe, p['att'])
    parts = _sc_scatter(y, dst_bin, zeros_acc)
    den = jax.ops.segment_sum(ex[:_E, 0], dst_pad[:_E],
                              num_segments=NUM_NODES)
    return _gat_combine(xl, xr, pe_loop, p['att'], parts, den,
                        p['bias'], pre)


def _gat_seg(xf, mm, p):
    N = xf.shape[0]
    mask = mm != 0.0
    cnt = jnp.sum(mask.astype(jnp.float32), axis=0)
    loop_attr = jnp.sum(mm, axis=0) / jnp.maximum(cnt, 1.0)
    xl = _mm(xf, p['Wl'], p['bl'])
    xr = _mm(xf, p['Wr'], p['br'])
    We = p['We'][0]
    s = xl[:, None, :] + xr[None, :, :] + mm[:, :, None] * We[None, None, :]
    m = jnp.where(s > 0, s, 0.2 * s)
    alpha = jnp.einsum('ijh,h->ij', m, p['att'])
    alpha = jnp.where(mask, alpha, -jnp.inf)
    sl = xl + xr + loop_attr[:, None] * We[None, :]
    ml = jnp.where(sl > 0, sl, 0.2 * sl)
    aloop = ml @ p['att']
    amax = jnp.maximum(jnp.max(jnp.where(mask, alpha, -jnp.inf), axis=0), aloop)
    ex = jnp.where(mask, jnp.exp(alpha - amax[None, :]), 0.0)
    exl = jnp.exp(aloop - amax)
    den = jnp.sum(ex, axis=0) + exl + 1e-16
    out = (ex.T @ xl + exl[:, None] * xl) / den[:, None]
    return out + p['bias']


def kernel(x, attr, segment_edge, segment_node, params, s_index, e_index,
           edge_index, cur_t, cur_w):
    p = params
    mm = segment_node @ segment_node.T
    src, dst = edge_index[0], edge_index[1]

    mask_idx = (jnp.sum(x, axis=1, keepdims=True) != XMIN * 4).astype(jnp.float32)
    ratio = 0.8 + 0.4 * jax.random.uniform(jax.random.key(1))
    drop_idx = (jax.random.uniform(jax.random.key(2), (x.shape[0], 1)) > 0.3
                ).astype(jnp.float32)
    x_norm = ((x - XMIN) / (XMAX - XMIN) * ratio * drop_idx).T
    h = _mm(x_norm, p['fc1_W'], p['fc1_b'], act='gelu')
    mu = _mm(h, p['fc2_W'], p['fc2_b'])
    log_var = _mm(h, p['fc3_W'], p['fc3_b'])
    eps = jax.random.normal(jax.random.key(3), mu.shape, jnp.float32)
    z = mu + eps * jnp.exp(log_var / 2)
    hz = _mm(z, p['fc4_W'], p['fc4_b'], act='gelu')
    x_rec = _mm(hz, p['fc5_W'], p['fc5_b'])
    x_rec = (x_rec / ratio).T
    x_rec = x_rec * (XMAX - XMIN) + XMIN
    x_rec1 = mask_idx * x + (1 - mask_idx) * x_rec

    ones = jnp.ones(dst.shape[0], jnp.float32)
    cnt = jax.ops.segment_sum(ones, dst, num_segments=NUM_NODES)
    am16 = (jax.ops.segment_sum(attr, dst, num_segments=NUM_NODES)
            / jnp.maximum(cnt, 1.0)[:, None])

    npad = _EP - _E
    src_pad = jnp.pad(src, (0, npad)).astype(jnp.int32)
    dst_pad = jnp.pad(dst, (0, npad)).astype(jnp.int32)
    dst_bin = jnp.pad(dst, (0, npad),
                      constant_values=NUM_NODES).astype(jnp.int32)
    attr_pad = jnp.pad(attr, ((0, npad), (0, 0)))
    zeros_acc = jnp.zeros((_NBIN, _HE), jnp.float32)

    ne = p['node_embed']
    pre = ne
    for name in ('conv1_0', 'conv1_1'):
        ne = _gat_big(ne, src_pad, dst_pad, dst_bin, attr_pad, am16, p[name],
                      p['attr_lin_W'], p['attr_lin_b'], pre, zeros_acc)
    data = _mm(x_rec1, p['node_lin_W'], p['node_lin_b'], act='gelu')
    pre = data
    for name in ('conv2_0', 'conv2_1'):
        data = _gat_big(data, src_pad, dst_pad, dst_bin, attr_pad, am16,
                        p[name], p['attr_lin_W'], p['attr_lin_b'], pre,
                        zeros_acc)

    time_embed = p['time_embed'][cur_t]
    week_embed = p['week_embed'][cur_w]
    seg_embed = p['segment_embed'][s_index]
    h1 = _mm(attr, p['attr1_W1'], p['attr1_b1'], act='leaky01')
    attr2 = (_mm(_mm(segment_edge, h1), p['attr1_W2'])
             + jnp.sum(segment_edge, axis=1, keepdims=True)
             * p['attr1_b2'][None, :])
    embed = jnp.take(p['edge_embed'], e_index, axis=0)
    attr3 = _mm(segment_edge, embed)
    xnh = _mm(segment_node, jnp.concatenate([data, ne], axis=1))
    x2, x1 = xnh[:, :128], xnh[:, 128:]
    xf = jnp.concatenate([seg_embed, time_embed, attr2, attr3, x2, x1,
                          week_embed], axis=1)
    xf1 = _gelu(_gat_seg(xf, mm, p['conv3']))
    for name in ('lin0', 'lin1', 'lin2'):
        xf = _mm(xf, p[name + '_W'], p[name + '_b'], act='gelu')
    xf = _mm(jnp.concatenate([xf, xf1], axis=1), p['lin3_W'], p['lin3_b'])
    out = jax.nn.sigmoid(xf) * 3600.0
    return out, x_rec


# speedup vs baseline: 2.9590x; 1.0065x over previous
import functools
import math

import jax
import jax.numpy as jnp
import numpy as np
from jax import lax
from jax.experimental import pallas as pl
from jax.experimental.pallas import tpu as pltpu
from jax.experimental.pallas import tpu_sc as plsc

XMIN = -1.21
XMAX = 23.91
NUM_NODES = 10000
SQRT_2_PI = float(np.sqrt(2.0 / np.pi))

_NC, _NS = 2, 16
_NW = _NC * _NS
_E = 160000
_EP = 163840
_PW = _EP // _NW
_CH = 64
_NCH = _PW // _CH
_NBIN = 10112
_RPS = _NBIN // _NS
_HE = 128
_SC_MESH = plsc.VectorSubcoreMesh(core_axis_name="c", subcore_axis_name="s")


_GNB = 8
_GIT = 2 * _NCH // _GNB


@functools.partial(
    pl.kernel,
    out_type=[jax.ShapeDtypeStruct((_EP, 128), jnp.float32),
              jax.ShapeDtypeStruct((_EP, 128), jnp.float32)],
    mesh=_SC_MESH,
    scratch_types=([pltpu.VMEM((_CH,), jnp.int32)] * _GNB
                   + [pltpu.VMEM((_CH, 128), jnp.float32)] * _GNB
                   + [pltpu.SemaphoreType.DMA] * _GNB),
)
def _sc_gather2(xl_hbm, xr_hbm, src_hbm, dst_hbm, o1_hbm, o2_hbm, *scr):
    idxs = scr[:_GNB]
    rows = scr[_GNB:2 * _GNB]
    sems = scr[2 * _GNB:]
    wid = lax.axis_index("s") * _NC + lax.axis_index("c")
    base = wid * _PW
    tabs = [xl_hbm, xr_hbm]
    srcs = [src_hbm, dst_hbm]
    outs = [o1_hbm, o2_hbm]

    for b in range(_GNB):
        off = base + (b // 2) * _CH
        pltpu.sync_copy(srcs[b % 2].at[pl.ds(off, _CH)], idxs[b])
        pltpu.async_copy(tabs[b % 2].at[idxs[b]], rows[b], sems[b])

    def body(k, carry):
        for b in range(_GNB):
            off = base + (b // 2 + (_GNB // 2) * k) * _CH
            pltpu.make_async_copy(tabs[b % 2].at[idxs[b]], rows[b],
                                  sems[b]).wait()
            pltpu.sync_copy(rows[b], outs[b % 2].at[pl.ds(off, _CH)])

            @pl.when(k + 1 < _GIT)
            def _():
                off2 = off + (_GNB // 2) * _CH
                pltpu.sync_copy(srcs[b % 2].at[pl.ds(off2, _CH)], idxs[b])
                pltpu.async_copy(tabs[b % 2].at[idxs[b]], rows[b], sems[b])
        return carry

    lax.fori_loop(0, _GIT, body, 0)


_SNB = 2


@functools.partial(
    pl.kernel,
    out_type=jax.ShapeDtypeStruct((_NC, _NBIN, _HE), jnp.float32),
    mesh=_SC_MESH,
    scratch_types=([pltpu.VMEM((_CH,), jnp.int32)] * _SNB
                   + [pltpu.VMEM((_CH, _HE), jnp.float32)] * _SNB
                   + [pltpu.SemaphoreType.DMA] * _SNB
                   + [pltpu.VMEM_SHARED((_NBIN, _HE), jnp.float32)]),
)
def _sc_scatter(y_hbm, dst_hbm, z_hbm, out_hbm, *scr):
    idxs = scr[:_SNB]
    rows = scr[_SNB:2 * _SNB]
    sems = scr[2 * _SNB:3 * _SNB]
    acc_sh = scr[3 * _SNB]
    cid = lax.axis_index("c")
    sid = lax.axis_index("s")
    wid = sid * _NC + cid
    r0 = sid * _RPS
    base = wid * _PW

    for b in range(_SNB):
        off = base + b * _CH
        pltpu.sync_copy(dst_hbm.at[pl.ds(off, _CH)], idxs[b])
        pltpu.async_copy(y_hbm.at[pl.ds(off, _CH)], rows[b], sems[b])

    pltpu.sync_copy(z_hbm.at[pl.ds(r0, _RPS)], acc_sh.at[pl.ds(r0, _RPS)])
    plsc.subcore_barrier()

    def body(k, carry):
        for b in range(_SNB):
            off = base + (b + _SNB * k) * _CH
            pltpu.make_async_copy(y_hbm.at[pl.ds(off, _CH)], rows[b],
                                  sems[b]).wait()
            pltpu.sync_copy(rows[b], acc_sh.at[idxs[b]], add=True)

            @pl.when(k + 1 < _NCH // _SNB)
            def _():
                off2 = off + _SNB * _CH
                pltpu.sync_copy(dst_hbm.at[pl.ds(off2, _CH)], idxs[b])
                pltpu.async_copy(y_hbm.at[pl.ds(off2, _CH)], rows[b], sems[b])
        return carry

    lax.fori_loop(0, _NCH // _SNB, body, 0)
    plsc.subcore_barrier()
    pltpu.sync_copy(acc_sh.at[pl.ds(r0, _RPS)],
                    out_hbm.at[cid].at[pl.ds(r0, _RPS)])


def _gelu(x):
    return 0.5 * x * (1.0 + jnp.tanh(SQRT_2_PI * (x + 0.044715 * x * x * x)))


def _act(r, act):
    if act == 'gelu':
        return _gelu(r)
    if act == 'leaky2':
        return jnp.where(r > 0, r, 0.2 * r)
    if act == 'leaky01':
        return jnp.where(r > 0, r, 0.01 * r)
    return r


def _rup(v, m):
    return ((v + m - 1) // m) * m


def _mm_body(a_ref, b_ref, bias_ref, o_ref, acc_ref, *, nk, act):
    k = pl.program_id(2)

    @pl.when(k == 0)
    def _():
        acc_ref[...] = jnp.zeros_like(acc_ref)

    acc_ref[...] += jnp.dot(a_ref[...], b_ref[...],
                            preferred_element_type=jnp.float32)

    @pl.when(k == nk - 1)
    def _():
        r = acc_ref[...] + bias_ref[...]
        o_ref[...] = _act(r, act)


@functools.partial(jax.jit, static_argnames=('act',))
def _mm(a, b, bias=None, act=None):
    M, K = a.shape
    K2, N = b.shape
    assert K == K2
    if bias is None:
        bias = jnp.zeros((N,), jnp.float32)
    bm = _rup(M, 8) if M < 256 else 256
    Mp = _rup(M, bm)
    Np = _rup(N, 128)
    bn = min(Np, 512)
    Np = _rup(Np, bn)
    if K <= 2048:
        bk = _rup(K, 8)
    else:
        bk = 3200 if K % 3200 == 0 else 2560
    Kp = _rup(K, bk)
    a = jnp.pad(a, ((0, Mp - M), (0, Kp - K)))
    b = jnp.pad(b, ((0, Kp - K), (0, Np - N)))
    bias = jnp.pad(bias, (0, Np - N)).reshape(1, Np)
    nk = Kp // bk
    out = pl.pallas_call(
        functools.partial(_mm_body, nk=nk, act=act),
        grid=(Mp // bm, Np // bn, nk),
        in_specs=[
            pl.BlockSpec((bm, bk), lambda m, n, k: (m, k)),
            pl.BlockSpec((bk, bn), lambda m, n, k: (k, n)),
            pl.BlockSpec((1, bn), lambda m, n, k: (0, n)),
        ],
        out_specs=pl.BlockSpec((bm, bn), lambda m, n, k: (m, n)),
        out_shape=jax.ShapeDtypeStruct((Mp, Np), jnp.float32),
        scratch_shapes=[pltpu.VMEM((bm, bn), jnp.float32)],
        compiler_params=pltpu.CompilerParams(
            dimension_semantics=("parallel", "parallel", "arbitrary")),
    )(a, b, bias)
    return out[:M, :N]


def _edge_y_body(xls_ref, xrd_ref, pe_ref, att_ref, y_ref, ex_ref, *, bm):
    xls = xls_ref[...]
    s = xls + xrd_ref[...] + pe_ref[...]
    m = jnp.where(s > 0, s, 0.2 * s)
    ex = jnp.exp(jnp.sum(m * att_ref[...], axis=1, keepdims=True))
    row = (pl.program_id(0) * bm
           + lax.broadcasted_iota(jnp.int32, (bm, 1), 0))
    ex = jnp.where(row < _E, ex, 0.0)
    y_ref[...] = ex * xls
    ex_ref[...] = ex


@jax.jit
def _edge_y(xlsrc, xrdst, pe, att):
    E, H = xlsrc.shape
    bm = 1280
    return pl.pallas_call(
        functools.partial(_edge_y_body, bm=bm),
        grid=(E // bm,),
        in_specs=[
            pl.BlockSpec((bm, H), lambda m: (m, 0)),
            pl.BlockSpec((bm, H), lambda m: (m, 0)),
            pl.BlockSpec((bm, H), lambda m: (m, 0)),
            pl.BlockSpec((1, H), lambda m: (0, 0)),
        ],
        out_specs=[
            pl.BlockSpec((bm, _HE), lambda m: (m, 0)),
            pl.BlockSpec((bm, 1), lambda m: (m, 0)),
        ],
        out_shape=[
            jax.ShapeDtypeStruct((E, _HE), jnp.float32),
            jax.ShapeDtypeStruct((E, 1), jnp.float32),
        ],
        compiler_params=pltpu.CompilerParams(
            dimension_semantics=("parallel",)),
    )(xlsrc, xrdst, pe, att.reshape(1, H))


def _combine_body(xl_ref, xr_ref, pel_ref, att_ref, p0_ref, p1_ref,
                  den_ref, bias_ref, pre_ref, o_ref):
    xl = xl_ref[...]
    s = xl + xr_ref[...] + pel_ref[...]
    m = jnp.where(s > 0, s, 0.2 * s)
    exl = jnp.exp(jnp.sum(m * att_ref[...], axis=1, keepdims=True))
    num = p0_ref[0] + p1_ref[0]
    r = (num + exl * xl) / (den_ref[...] + exl + 1e-16)
    r = r + bias_ref[...]
    o_ref[...] = _gelu(r) + pre_ref[...]


@jax.jit
def _gat_combine(xl, xr, pe_loop, att, parts, den, bias, pre):
    N, H = xl.shape
    bm = 1000
    return pl.pallas_call(
        _combine_body,
        grid=(N // bm,),
        in_specs=[
            pl.BlockSpec((bm, H), lambda m: (m, 0)),
            pl.BlockSpec((bm, H), lambda m: (m, 0)),
            pl.BlockSpec((bm, H), lambda m: (m, 0)),
            pl.BlockSpec((1, H), lambda m: (0, 0)),
            pl.BlockSpec((1, bm, _HE), lambda m: (0, m, 0)),
            pl.BlockSpec((1, bm, _HE), lambda m: (1, m, 0)),
            pl.BlockSpec((bm, 1), lambda m: (m, 0)),
            pl.BlockSpec((1, H), lambda m: (0, 0)),
            pl.BlockSpec((bm, H), lambda m: (m, 0)),
        ],
        out_specs=pl.BlockSpec((bm, H), lambda m: (m, 0)),
        out_shape=jax.ShapeDtypeStruct((N, H), jnp.float32),
        compiler_params=pltpu.CompilerParams(
            dimension_semantics=("parallel",)),
    )(xl, xr, pe_loop, att.reshape(1, H), parts, parts,
      den.reshape(N, 1), bias.reshape(1, H), pre)


def _gat_big(xin, src_pad, dst_pad, dst_bin, attr_pad, am16, p,
             lin_W, lin_b, pre, zeros_acc):
    Wc = _mm(lin_W, p['We'])
    bc = lin_b @ p['We']
    xl = _mm(xin, p['Wl'], p['bl'])
    xr = _mm(xin, p['Wr'], p['br'])
    pe = _mm(attr_pad, Wc, bc)
    pe_loop = _mm(am16, Wc, bc)
    xlsrc, xrdst = _sc_gather2(xl, xr, src_pad, dst_pad)
    y, ex = _edge_y(xlsrc, xrdst, pe, p['att'])
    parts = _sc_scatter(y, dst_bin, zeros_acc)
    den = jax.ops.segment_sum(ex[:_E, 0], dst_pad[:_E],
                              num_segments=NUM_NODES)
    return _gat_combine(xl, xr, pe_loop, p['att'], parts, den,
                        p['bias'], pre)


def _gat_seg(xf, mm, p):
    N = xf.shape[0]
    mask = mm != 0.0
    cnt = jnp.sum(mask.astype(jnp.float32), axis=0)
    loop_attr = jnp.sum(mm, axis=0) / jnp.maximum(cnt, 1.0)
    xl = _mm(xf, p['Wl'], p['bl'])
    xr = _mm(xf, p['Wr'], p['br'])
    We = p['We'][0]
    s = xl[:, None, :] + xr[None, :, :] + mm[:, :, None] * We[None, None, :]
    m = jnp.where(s > 0, s, 0.2 * s)
    alpha = jnp.einsum('ijh,h->ij', m, p['att'])
    alpha = jnp.where(mask, alpha, -jnp.inf)
    sl = xl + xr + loop_attr[:, None] * We[None, :]
    ml = jnp.where(sl > 0, sl, 0.2 * sl)
    aloop = ml @ p['att']
    amax = jnp.maximum(jnp.max(jnp.where(mask, alpha, -jnp.inf), axis=0), aloop)
    ex = jnp.where(mask, jnp.exp(alpha - amax[None, :]), 0.0)
    exl = jnp.exp(aloop - amax)
    den = jnp.sum(ex, axis=0) + exl + 1e-16
    out = (ex.T @ xl + exl[:, None] * xl) / den[:, None]
    return out + p['bias']


def kernel(x, attr, segment_edge, segment_node, params, s_index, e_index,
           edge_index, cur_t, cur_w):
    p = params
    mm = segment_node @ segment_node.T
    src, dst = edge_index[0], edge_index[1]

    mask_idx = (jnp.sum(x, axis=1, keepdims=True) != XMIN * 4).astype(jnp.float32)
    ratio = 0.8 + 0.4 * jax.random.uniform(jax.random.key(1))
    drop_idx = (jax.random.uniform(jax.random.key(2), (x.shape[0], 1)) > 0.3
                ).astype(jnp.float32)
    x_norm = ((x - XMIN) / (XMAX - XMIN) * ratio * drop_idx).T
    h = _mm(x_norm, p['fc1_W'], p['fc1_b'], act='gelu')
    mu = _mm(h, p['fc2_W'], p['fc2_b'])
    log_var = _mm(h, p['fc3_W'], p['fc3_b'])
    eps = jax.random.normal(jax.random.key(3), mu.shape, jnp.float32)
    z = mu + eps * jnp.exp(log_var / 2)
    hz = _mm(z, p['fc4_W'], p['fc4_b'], act='gelu')
    x_rec = _mm(hz, p['fc5_W'], p['fc5_b'])
    x_rec = (x_rec / ratio).T
    x_rec = x_rec * (XMAX - XMIN) + XMIN
    x_rec1 = mask_idx * x + (1 - mask_idx) * x_rec

    ones = jnp.ones(dst.shape[0], jnp.float32)
    cnt = jax.ops.segment_sum(ones, dst, num_segments=NUM_NODES)
    am16 = (jax.ops.segment_sum(attr, dst, num_segments=NUM_NODES)
            / jnp.maximum(cnt, 1.0)[:, None])

    npad = _EP - _E
    src_pad = jnp.pad(src, (0, npad)).astype(jnp.int32)
    dst_pad = jnp.pad(dst, (0, npad)).astype(jnp.int32)
    dst_bin = jnp.pad(dst, (0, npad),
                      constant_values=NUM_NODES).astype(jnp.int32)
    attr_pad = jnp.pad(attr, ((0, npad), (0, 0)))
    zeros_acc = jnp.zeros((_NBIN, _HE), jnp.float32)

    ne = p['node_embed']
    pre = ne
    for name in ('conv1_0', 'conv1_1'):
        ne = _gat_big(ne, src_pad, dst_pad, dst_bin, attr_pad, am16, p[name],
                      p['attr_lin_W'], p['attr_lin_b'], pre, zeros_acc)
    data = _mm(x_rec1, p['node_lin_W'], p['node_lin_b'], act='gelu')
    pre = data
    for name in ('conv2_0', 'conv2_1'):
        data = _gat_big(data, src_pad, dst_pad, dst_bin, attr_pad, am16,
                        p[name], p['attr_lin_W'], p['attr_lin_b'], pre,
                        zeros_acc)

    time_embed = p['time_embed'][cur_t]
    week_embed = p['week_embed'][cur_w]
    seg_embed = p['segment_embed'][s_index]
    h1 = _mm(attr, p['attr1_W1'], p['attr1_b1'], act='leaky01')
    attr2 = (_mm(_mm(segment_edge, h1), p['attr1_W2'])
             + jnp.sum(segment_edge, axis=1, keepdims=True)
             * p['attr1_b2'][None, :])
    embed = jnp.take(p['edge_embed'], e_index, axis=0)
    attr3 = _mm(segment_edge, embed)
    xnh = _mm(segment_node, jnp.concatenate([data, ne], axis=1))
    x2, x1 = xnh[:, :128], xnh[:, 128:]
    xf = jnp.concatenate([seg_embed, time_embed, attr2, attr3, x2, x1,
                          week_embed], axis=1)
    xf1 = _gelu(_gat_seg(xf, mm, p['conv3']))
    for name in ('lin0', 'lin1', 'lin2'):
        xf = _mm(xf, p[name + '_W'], p[name + '_b'], act='gelu')
    xf = _mm(jnp.concatenate([xf, xf1], axis=1), p['lin3_W'], p['lin3_b'])
    out = jax.nn.sigmoid(xf) * 3600.0
    return out, x_rec


# speedup vs baseline: 2.9915x; 1.0110x over previous
import functools
import math

import jax
import jax.numpy as jnp
import numpy as np
from jax import lax
from jax.experimental import pallas as pl
from jax.experimental.pallas import tpu as pltpu
from jax.experimental.pallas import tpu_sc as plsc

XMIN = -1.21
XMAX = 23.91
NUM_NODES = 10000
SQRT_2_PI = float(np.sqrt(2.0 / np.pi))

_NC, _NS = 2, 16
_NW = _NC * _NS
_E = 160000
_EP = 163840
_PW = _EP // _NW
_CH = 64
_NCH = _PW // _CH
_NBIN = 10112
_RPS = _NBIN // _NS
_HE = 128
_SC_MESH = plsc.VectorSubcoreMesh(core_axis_name="c", subcore_axis_name="s")


_GNB = 8
_GIT = 2 * _NCH // _GNB


@functools.partial(
    pl.kernel,
    out_type=[jax.ShapeDtypeStruct((_EP, 128), jnp.float32),
              jax.ShapeDtypeStruct((_EP, 128), jnp.float32)],
    mesh=_SC_MESH,
    scratch_types=([pltpu.VMEM((_CH,), jnp.int32)] * _GNB
                   + [pltpu.VMEM((_CH, 128), jnp.float32)] * _GNB
                   + [pltpu.SemaphoreType.DMA] * _GNB),
)
def _sc_gather2(xl_hbm, xr_hbm, src_hbm, dst_hbm, o1_hbm, o2_hbm, *scr):
    idxs = scr[:_GNB]
    rows = scr[_GNB:2 * _GNB]
    sems = scr[2 * _GNB:]
    wid = lax.axis_index("s") * _NC + lax.axis_index("c")
    base = wid * _PW
    tabs = [xl_hbm, xr_hbm]
    srcs = [src_hbm, dst_hbm]
    outs = [o1_hbm, o2_hbm]

    for b in range(_GNB):
        off = base + (b // 2) * _CH
        pltpu.sync_copy(srcs[b % 2].at[pl.ds(off, _CH)], idxs[b])
        pltpu.async_copy(tabs[b % 2].at[idxs[b]], rows[b], sems[b])

    def body(k, carry):
        for b in range(_GNB):
            off = base + (b // 2 + (_GNB // 2) * k) * _CH
            pltpu.make_async_copy(tabs[b % 2].at[idxs[b]], rows[b],
                                  sems[b]).wait()
            pltpu.sync_copy(rows[b], outs[b % 2].at[pl.ds(off, _CH)])

            @pl.when(k + 1 < _GIT)
            def _():
                off2 = off + (_GNB // 2) * _CH
                pltpu.sync_copy(srcs[b % 2].at[pl.ds(off2, _CH)], idxs[b])
                pltpu.async_copy(tabs[b % 2].at[idxs[b]], rows[b], sems[b])
        return carry

    lax.fori_loop(0, _GIT, body, 0)


_G4IT = 4 * _NCH // _GNB


@functools.partial(
    pl.kernel,
    out_type=[jax.ShapeDtypeStruct((_EP, 128), jnp.float32)] * 4,
    mesh=_SC_MESH,
    scratch_types=([pltpu.VMEM((_CH,), jnp.int32)] * _GNB
                   + [pltpu.VMEM((_CH, 128), jnp.float32)] * _GNB
                   + [pltpu.SemaphoreType.DMA] * _GNB),
)
def _sc_gather4(t0, t1, t2, t3, src_hbm, dst_hbm, o0, o1, o2, o3, *scr):
    idxs = scr[:_GNB]
    rows = scr[_GNB:2 * _GNB]
    sems = scr[2 * _GNB:]
    wid = lax.axis_index("s") * _NC + lax.axis_index("c")
    base = wid * _PW
    tabs = [t0, t1, t2, t3]
    srcs = [src_hbm, dst_hbm, src_hbm, dst_hbm]
    outs = [o0, o1, o2, o3]
    nst = _GNB // 4

    for b in range(_GNB):
        off = base + (b // 4) * _CH
        pltpu.sync_copy(srcs[b % 4].at[pl.ds(off, _CH)], idxs[b])
        pltpu.async_copy(tabs[b % 4].at[idxs[b]], rows[b], sems[b])

    def body(k, carry):
        for b in range(_GNB):
            off = base + (b // 4 + nst * k) * _CH
            pltpu.make_async_copy(tabs[b % 4].at[idxs[b]], rows[b],
                                  sems[b]).wait()
            pltpu.sync_copy(rows[b], outs[b % 4].at[pl.ds(off, _CH)])

            @pl.when(k + 1 < _G4IT)
            def _():
                off2 = off + nst * _CH
                pltpu.sync_copy(srcs[b % 4].at[pl.ds(off2, _CH)], idxs[b])
                pltpu.async_copy(tabs[b % 4].at[idxs[b]], rows[b], sems[b])
        return carry

    lax.fori_loop(0, _G4IT, body, 0)


_SNB = 2


@functools.partial(
    pl.kernel,
    out_type=jax.ShapeDtypeStruct((_NC, _NBIN, _HE), jnp.float32),
    mesh=_SC_MESH,
    scratch_types=([pltpu.VMEM((_CH,), jnp.int32)] * _SNB
                   + [pltpu.VMEM((_CH, _HE), jnp.float32)] * _SNB
                   + [pltpu.SemaphoreType.DMA] * _SNB
                   + [pltpu.VMEM_SHARED((_NBIN, _HE), jnp.float32)]),
)
def _sc_scatter(y_hbm, dst_hbm, z_hbm, out_hbm, *scr):
    idxs = scr[:_SNB]
    rows = scr[_SNB:2 * _SNB]
    sems = scr[2 * _SNB:3 * _SNB]
    acc_sh = scr[3 * _SNB]
    cid = lax.axis_index("c")
    sid = lax.axis_index("s")
    wid = sid * _NC + cid
    r0 = sid * _RPS
    base = wid * _PW

    for b in range(_SNB):
        off = base + b * _CH
        pltpu.sync_copy(dst_hbm.at[pl.ds(off, _CH)], idxs[b])
        pltpu.async_copy(y_hbm.at[pl.ds(off, _CH)], rows[b], sems[b])

    pltpu.sync_copy(z_hbm.at[pl.ds(r0, _RPS)], acc_sh.at[pl.ds(r0, _RPS)])
    plsc.subcore_barrier()

    def body(k, carry):
        for b in range(_SNB):
            off = base + (b + _SNB * k) * _CH
            pltpu.make_async_copy(y_hbm.at[pl.ds(off, _CH)], rows[b],
                                  sems[b]).wait()
            pltpu.sync_copy(rows[b], acc_sh.at[idxs[b]], add=True)

            @pl.when(k + 1 < _NCH // _SNB)
            def _():
                off2 = off + _SNB * _CH
                pltpu.sync_copy(dst_hbm.at[pl.ds(off2, _CH)], idxs[b])
                pltpu.async_copy(y_hbm.at[pl.ds(off2, _CH)], rows[b], sems[b])
        return carry

    lax.fori_loop(0, _NCH // _SNB, body, 0)
    plsc.subcore_barrier()
    pltpu.sync_copy(acc_sh.at[pl.ds(r0, _RPS)],
                    out_hbm.at[cid].at[pl.ds(r0, _RPS)])


def _gelu(x):
    return 0.5 * x * (1.0 + jnp.tanh(SQRT_2_PI * (x + 0.044715 * x * x * x)))


def _act(r, act):
    if act == 'gelu':
        return _gelu(r)
    if act == 'leaky2':
        return jnp.where(r > 0, r, 0.2 * r)
    if act == 'leaky01':
        return jnp.where(r > 0, r, 0.01 * r)
    return r


def _rup(v, m):
    return ((v + m - 1) // m) * m


def _mm_body(a_ref, b_ref, bias_ref, o_ref, acc_ref, *, nk, act):
    k = pl.program_id(2)

    @pl.when(k == 0)
    def _():
        acc_ref[...] = jnp.zeros_like(acc_ref)

    acc_ref[...] += jnp.dot(a_ref[...], b_ref[...],
                            preferred_element_type=jnp.float32)

    @pl.when(k == nk - 1)
    def _():
        r = acc_ref[...] + bias_ref[...]
        o_ref[...] = _act(r, act)


@functools.partial(jax.jit, static_argnames=('act',))
def _mm(a, b, bias=None, act=None):
    M, K = a.shape
    K2, N = b.shape
    assert K == K2
    if bias is None:
        bias = jnp.zeros((N,), jnp.float32)
    bm = _rup(M, 8) if M < 256 else 256
    Mp = _rup(M, bm)
    Np = _rup(N, 128)
    bn = min(Np, 512)
    Np = _rup(Np, bn)
    if K <= 2048:
        bk = _rup(K, 8)
    else:
        bk = 3200 if K % 3200 == 0 else 2560
    Kp = _rup(K, bk)
    a = jnp.pad(a, ((0, Mp - M), (0, Kp - K)))
    b = jnp.pad(b, ((0, Kp - K), (0, Np - N)))
    bias = jnp.pad(bias, (0, Np - N)).reshape(1, Np)
    nk = Kp // bk
    out = pl.pallas_call(
        functools.partial(_mm_body, nk=nk, act=act),
        grid=(Mp // bm, Np // bn, nk),
        in_specs=[
            pl.BlockSpec((bm, bk), lambda m, n, k: (m, k)),
            pl.BlockSpec((bk, bn), lambda m, n, k: (k, n)),
            pl.BlockSpec((1, bn), lambda m, n, k: (0, n)),
        ],
        out_specs=pl.BlockSpec((bm, bn), lambda m, n, k: (m, n)),
        out_shape=jax.ShapeDtypeStruct((Mp, Np), jnp.float32),
        scratch_shapes=[pltpu.VMEM((bm, bn), jnp.float32)],
        compiler_params=pltpu.CompilerParams(
            dimension_semantics=("parallel", "parallel", "arbitrary")),
    )(a, b, bias)
    return out[:M, :N]


def _edge_y_body(xls_ref, xrd_ref, pe_ref, att_ref, y_ref, ex_ref, *, bm):
    xls = xls_ref[...]
    s = xls + xrd_ref[...] + pe_ref[...]
    m = jnp.where(s > 0, s, 0.2 * s)
    ex = jnp.exp(jnp.sum(m * att_ref[...], axis=1, keepdims=True))
    row = (pl.program_id(0) * bm
           + lax.broadcasted_iota(jnp.int32, (bm, 1), 0))
    ex = jnp.where(row < _E, ex, 0.0)
    y_ref[...] = ex * xls
    ex_ref[...] = ex


@jax.jit
def _edge_y(xlsrc, xrdst, pe, att):
    E, H = xlsrc.shape
    bm = 1280
    return pl.pallas_call(
        functools.partial(_edge_y_body, bm=bm),
        grid=(E // bm,),
        in_specs=[
            pl.BlockSpec((bm, H), lambda m: (m, 0)),
            pl.BlockSpec((bm, H), lambda m: (m, 0)),
            pl.BlockSpec((bm, H), lambda m: (m, 0)),
            pl.BlockSpec((1, H), lambda m: (0, 0)),
        ],
        out_specs=[
            pl.BlockSpec((bm, _HE), lambda m: (m, 0)),
            pl.BlockSpec((bm, 1), lambda m: (m, 0)),
        ],
        out_shape=[
            jax.ShapeDtypeStruct((E, _HE), jnp.float32),
            jax.ShapeDtypeStruct((E, 1), jnp.float32),
        ],
        compiler_params=pltpu.CompilerParams(
            dimension_semantics=("parallel",)),
    )(xlsrc, xrdst, pe, att.reshape(1, H))


def _combine_body(xl_ref, xr_ref, pel_ref, att_ref, p0_ref, p1_ref,
                  den_ref, bias_ref, pre_ref, o_ref):
    xl = xl_ref[...]
    s = xl + xr_ref[...] + pel_ref[...]
    m = jnp.where(s > 0, s, 0.2 * s)
    exl = jnp.exp(jnp.sum(m * att_ref[...], axis=1, keepdims=True))
    num = p0_ref[0] + p1_ref[0]
    r = (num + exl * xl) / (den_ref[...] + exl + 1e-16)
    r = r + bias_ref[...]
    o_ref[...] = _gelu(r) + pre_ref[...]


@jax.jit
def _gat_combine(xl, xr, pe_loop, att, parts, den, bias, pre):
    N, H = xl.shape
    bm = 1000
    return pl.pallas_call(
        _combine_body,
        grid=(N // bm,),
        in_specs=[
            pl.BlockSpec((bm, H), lambda m: (m, 0)),
            pl.BlockSpec((bm, H), lambda m: (m, 0)),
            pl.BlockSpec((bm, H), lambda m: (m, 0)),
            pl.BlockSpec((1, H), lambda m: (0, 0)),
            pl.BlockSpec((1, bm, _HE), lambda m: (0, m, 0)),
            pl.BlockSpec((1, bm, _HE), lambda m: (1, m, 0)),
            pl.BlockSpec((bm, 1), lambda m: (m, 0)),
            pl.BlockSpec((1, H), lambda m: (0, 0)),
            pl.BlockSpec((bm, H), lambda m: (m, 0)),
        ],
        out_specs=pl.BlockSpec((bm, H), lambda m: (m, 0)),
        out_shape=jax.ShapeDtypeStruct((N, H), jnp.float32),
        compiler_params=pltpu.CompilerParams(
            dimension_semantics=("parallel",)),
    )(xl, xr, pe_loop, att.reshape(1, H), parts, parts,
      den.reshape(N, 1), bias.reshape(1, H), pre)


def _gat_big(xin, src_pad, dst_pad, dst_bin, attr_pad, am16, p,
             lin_W, lin_b, pre, zeros_acc):
    Wc = _mm(lin_W, p['We'])
    bc = lin_b @ p['We']
    xl = _mm(xin, p['Wl'], p['bl'])
    xr = _mm(xin, p['Wr'], p['br'])
    pe = _mm(attr_pad, Wc, bc)
    pe_loop = _mm(am16, Wc, bc)
    xlsrc, xrdst = _sc_gather2(xl, xr, src_pad, dst_pad)
    y, ex = _edge_y(xlsrc, xrdst, pe, p['att'])
    parts = _sc_scatter(y, dst_bin, zeros_acc)
    den = jax.ops.segment_sum(ex[:_E, 0], dst_pad[:_E],
                              num_segments=NUM_NODES)
    return _gat_combine(xl, xr, pe_loop, p['att'], parts, den,
                        p['bias'], pre)


def _gat_pair(x1, p1, x2, p2, src_pad, dst_pad, dst_bin, attr_pad, am16,
              lin_W, lin_b, pre1, pre2, zeros_acc):
    outs = []
    mms = []
    for p, x in ((p1, x1), (p2, x2)):
        Wc = _mm(lin_W, p['We'])
        bc = lin_b @ p['We']
        mms.append((_mm(x, p['Wl'], p['bl']), _mm(x, p['Wr'], p['br']),
                    _mm(attr_pad, Wc, bc), _mm(am16, Wc, bc)))
    (xl1, xr1, pe1, pel1), (xl2, xr2, pe2, pel2) = mms
    xls1, xrd1, xls2, xrd2 = _sc_gather4(xl1, xr1, xl2, xr2,
                                         src_pad, dst_pad)
    y1, ex1 = _edge_y(xls1, xrd1, pe1, p1['att'])
    y2, ex2 = _edge_y(xls2, xrd2, pe2, p2['att'])
    parts1 = _sc_scatter(y1, dst_bin, zeros_acc)
    parts2 = _sc_scatter(y2, dst_bin, zeros_acc)
    den1 = jax.ops.segment_sum(ex1[:_E, 0], dst_pad[:_E],
                               num_segments=NUM_NODES)
    den2 = jax.ops.segment_sum(ex2[:_E, 0], dst_pad[:_E],
                               num_segments=NUM_NODES)
    o1 = _gat_combine(xl1, xr1, pel1, p1['att'], parts1, den1,
                      p1['bias'], pre1)
    o2 = _gat_combine(xl2, xr2, pel2, p2['att'], parts2, den2,
                      p2['bias'], pre2)
    return o1, o2


def _gat_seg(xf, mm, p):
    N = xf.shape[0]
    mask = mm != 0.0
    cnt = jnp.sum(mask.astype(jnp.float32), axis=0)
    loop_attr = jnp.sum(mm, axis=0) / jnp.maximum(cnt, 1.0)
    xl = _mm(xf, p['Wl'], p['bl'])
    xr = _mm(xf, p['Wr'], p['br'])
    We = p['We'][0]
    s = xl[:, None, :] + xr[None, :, :] + mm[:, :, None] * We[None, None, :]
    m = jnp.where(s > 0, s, 0.2 * s)
    alpha = jnp.einsum('ijh,h->ij', m, p['att'])
    alpha = jnp.where(mask, alpha, -jnp.inf)
    sl = xl + xr + loop_attr[:, None] * We[None, :]
    ml = jnp.where(sl > 0, sl, 0.2 * sl)
    aloop = ml @ p['att']
    amax = jnp.maximum(jnp.max(jnp.where(mask, alpha, -jnp.inf), axis=0), aloop)
    ex = jnp.where(mask, jnp.exp(alpha - amax[None, :]), 0.0)
    exl = jnp.exp(aloop - amax)
    den = jnp.sum(ex, axis=0) + exl + 1e-16
    out = (ex.T @ xl + exl[:, None] * xl) / den[:, None]
    return out + p['bias']


def kernel(x, attr, segment_edge, segment_node, params, s_index, e_index,
           edge_index, cur_t, cur_w):
    p = params
    mm = segment_node @ segment_node.T
    src, dst = edge_index[0], edge_index[1]

    mask_idx = (jnp.sum(x, axis=1, keepdims=True) != XMIN * 4).astype(jnp.float32)
    ratio = 0.8 + 0.4 * jax.random.uniform(jax.random.key(1))
    drop_idx = (jax.random.uniform(jax.random.key(2), (x.shape[0], 1)) > 0.3
                ).astype(jnp.float32)
    x_norm = ((x - XMIN) / (XMAX - XMIN) * ratio * drop_idx).T
    h = _mm(x_norm, p['fc1_W'], p['fc1_b'], act='gelu')
    mu = _mm(h, p['fc2_W'], p['fc2_b'])
    log_var = _mm(h, p['fc3_W'], p['fc3_b'])
    eps = jax.random.normal(jax.random.key(3), mu.shape, jnp.float32)
    z = mu + eps * jnp.exp(log_var / 2)
    hz = _mm(z, p['fc4_W'], p['fc4_b'], act='gelu')
    x_rec = _mm(hz, p['fc5_W'], p['fc5_b'])
    x_rec = (x_rec / ratio).T
    x_rec = x_rec * (XMAX - XMIN) + XMIN
    x_rec1 = mask_idx * x + (1 - mask_idx) * x_rec

    ones = jnp.ones(dst.shape[0], jnp.float32)
    cnt = jax.ops.segment_sum(ones, dst, num_segments=NUM_NODES)
    am16 = (jax.ops.segment_sum(attr, dst, num_segments=NUM_NODES)
            / jnp.maximum(cnt, 1.0)[:, None])

    npad = _EP - _E
    src_pad = jnp.pad(src, (0, npad)).astype(jnp.int32)
    dst_pad = jnp.pad(dst, (0, npad)).astype(jnp.int32)
    dst_bin = jnp.pad(dst, (0, npad),
                      constant_values=NUM_NODES).astype(jnp.int32)
    attr_pad = jnp.pad(attr, ((0, npad), (0, 0)))
    zeros_acc = jnp.zeros((_NBIN, _HE), jnp.float32)

    ne = p['node_embed']
    pre1 = ne
    data = _mm(x_rec1, p['node_lin_W'], p['node_lin_b'], act='gelu')
    pre2 = data
    for layer in ('0', '1'):
        ne, data = _gat_pair(ne, p['conv1_' + layer], data,
                             p['conv2_' + layer], src_pad, dst_pad, dst_bin,
                             attr_pad, am16, p['attr_lin_W'],
                             p['attr_lin_b'], pre1, pre2, zeros_acc)

    time_embed = p['time_embed'][cur_t]
    week_embed = p['week_embed'][cur_w]
    seg_embed = p['segment_embed'][s_index]
    h1 = _mm(attr, p['attr1_W1'], p['attr1_b1'], act='leaky01')
    attr2 = (_mm(_mm(segment_edge, h1), p['attr1_W2'])
             + jnp.sum(segment_edge, axis=1, keepdims=True)
             * p['attr1_b2'][None, :])
    embed = jnp.take(p['edge_embed'], e_index, axis=0)
    attr3 = _mm(segment_edge, embed)
    xnh = _mm(segment_node, jnp.concatenate([data, ne], axis=1))
    x2, x1 = xnh[:, :128], xnh[:, 128:]
    xf = jnp.concatenate([seg_embed, time_embed, attr2, attr3, x2, x1,
                          week_embed], axis=1)
    xf1 = _gelu(_gat_seg(xf, mm, p['conv3']))
    for name in ('lin0', 'lin1', 'lin2'):
        xf = _mm(xf, p[name + '_W'], p[name + '_b'], act='gelu')
    xf = _mm(jnp.concatenate([xf, xf1], axis=1), p['lin3_W'], p['lin3_b'])
    out = jax.nn.sigmoid(xf) * 3600.0
    return out, x_rec


# speedup vs baseline: 3.8523x; 1.2877x over previous
import functools
import math

import jax
import jax.numpy as jnp
import numpy as np
from jax import lax
from jax.experimental import pallas as pl
from jax.experimental.pallas import tpu as pltpu
from jax.experimental.pallas import tpu_sc as plsc

XMIN = -1.21
XMAX = 23.91
NUM_NODES = 10000
SQRT_2_PI = float(np.sqrt(2.0 / np.pi))

_NC, _NS = 2, 16
_NW = _NC * _NS
_E = 160000
_EP = 163840
_PW = _EP // _NW
_CH = 64
_NCH = _PW // _CH
_NBIN = 10112
_RPS = _NBIN // _NS
_HE = 128
_SC_MESH = plsc.VectorSubcoreMesh(core_axis_name="c", subcore_axis_name="s")


_GNB = 8
_GIT = 2 * _NCH // _GNB


_G4IT = 4 * _NCH // _GNB


@functools.partial(
    pl.kernel,
    out_type=[jax.ShapeDtypeStruct((_EP, 128), jnp.float32)] * 4,
    mesh=_SC_MESH,
    scratch_types=([pltpu.VMEM((_CH,), jnp.int32)] * _GNB
                   + [pltpu.VMEM((_CH, 128), jnp.float32)] * _GNB
                   + [pltpu.SemaphoreType.DMA] * _GNB),
)
def _sc_gather4(t0, t1, t2, t3, src_hbm, dst_hbm, o0, o1, o2, o3, *scr):
    idxs = scr[:_GNB]
    rows = scr[_GNB:2 * _GNB]
    sems = scr[2 * _GNB:]
    wid = lax.axis_index("s") * _NC + lax.axis_index("c")
    base = wid * _PW
    tabs = [t0, t1, t2, t3]
    srcs = [src_hbm, dst_hbm, src_hbm, dst_hbm]
    outs = [o0, o1, o2, o3]
    nst = _GNB // 4

    for b in range(_GNB):
        off = base + (b // 4) * _CH
        pltpu.sync_copy(srcs[b % 4].at[pl.ds(off, _CH)], idxs[b])
        pltpu.async_copy(tabs[b % 4].at[idxs[b]], rows[b], sems[b])

    def body(k, carry):
        for b in range(_GNB):
            off = base + (b // 4 + nst * k) * _CH
            pltpu.make_async_copy(tabs[b % 4].at[idxs[b]], rows[b],
                                  sems[b]).wait()
            pltpu.sync_copy(rows[b], outs[b % 4].at[pl.ds(off, _CH)])

            @pl.when(k + 1 < _G4IT)
            def _():
                off2 = off + nst * _CH
                pltpu.sync_copy(srcs[b % 4].at[pl.ds(off2, _CH)], idxs[b])
                pltpu.async_copy(tabs[b % 4].at[idxs[b]], rows[b], sems[b])
        return carry

    lax.fori_loop(0, _G4IT, body, 0)


_SNB = 2


@functools.partial(
    pl.kernel,
    out_type=jax.ShapeDtypeStruct((_NC, _NBIN, _HE), jnp.float32),
    mesh=_SC_MESH,
    scratch_types=([pltpu.VMEM((_CH,), jnp.int32)] * _SNB
                   + [pltpu.VMEM((_CH, _HE), jnp.float32)] * _SNB
                   + [pltpu.SemaphoreType.DMA] * _SNB
                   + [pltpu.VMEM_SHARED((_NBIN, _HE), jnp.float32)]),
)
def _sc_scatter(y_hbm, dst_hbm, z_hbm, out_hbm, *scr):
    idxs = scr[:_SNB]
    rows = scr[_SNB:2 * _SNB]
    sems = scr[2 * _SNB:3 * _SNB]
    acc_sh = scr[3 * _SNB]
    cid = lax.axis_index("c")
    sid = lax.axis_index("s")
    wid = sid * _NC + cid
    r0 = sid * _RPS
    base = wid * _PW

    for b in range(_SNB):
        off = base + b * _CH
        pltpu.sync_copy(dst_hbm.at[pl.ds(off, _CH)], idxs[b])
        pltpu.async_copy(y_hbm.at[pl.ds(off, _CH)], rows[b], sems[b])

    pltpu.sync_copy(z_hbm.at[pl.ds(r0, _RPS)], acc_sh.at[pl.ds(r0, _RPS)])
    plsc.subcore_barrier()

    def body(k, carry):
        for b in range(_SNB):
            off = base + (b + _SNB * k) * _CH
            pltpu.make_async_copy(y_hbm.at[pl.ds(off, _CH)], rows[b],
                                  sems[b]).wait()
            pltpu.sync_copy(rows[b], acc_sh.at[idxs[b]], add=True)

            @pl.when(k + 1 < _NCH // _SNB)
            def _():
                off2 = off + _SNB * _CH
                pltpu.sync_copy(dst_hbm.at[pl.ds(off2, _CH)], idxs[b])
                pltpu.async_copy(y_hbm.at[pl.ds(off2, _CH)], rows[b], sems[b])
        return carry

    lax.fori_loop(0, _NCH // _SNB, body, 0)
    plsc.subcore_barrier()
    pltpu.sync_copy(acc_sh.at[pl.ds(r0, _RPS)],
                    out_hbm.at[cid].at[pl.ds(r0, _RPS)])


def _gelu(x):
    return 0.5 * x * (1.0 + jnp.tanh(SQRT_2_PI * (x + 0.044715 * x * x * x)))


def _act(r, act):
    if act == 'gelu':
        return _gelu(r)
    if act == 'leaky2':
        return jnp.where(r > 0, r, 0.2 * r)
    if act == 'leaky01':
        return jnp.where(r > 0, r, 0.01 * r)
    return r


def _rup(v, m):
    return ((v + m - 1) // m) * m


def _mm_body(a_ref, b_ref, bias_ref, o_ref, acc_ref, *, nk, act):
    k = pl.program_id(2)

    @pl.when(k == 0)
    def _():
        acc_ref[...] = jnp.zeros_like(acc_ref)

    acc_ref[...] += jnp.dot(a_ref[...], b_ref[...],
                            preferred_element_type=jnp.float32)

    @pl.when(k == nk - 1)
    def _():
        r = acc_ref[...] + bias_ref[...]
        o_ref[...] = _act(r, act)


@functools.partial(jax.jit, static_argnames=('act',))
def _mm(a, b, bias=None, act=None):
    M, K = a.shape
    K2, N = b.shape
    assert K == K2
    if bias is None:
        bias = jnp.zeros((N,), jnp.float32)
    bm = _rup(M, 8) if M < 256 else 256
    Mp = _rup(M, bm)
    Np = _rup(N, 128)
    bn = min(Np, 512)
    Np = _rup(Np, bn)
    if K <= 2048:
        bk = _rup(K, 8)
    else:
        bk = 3200 if K % 3200 == 0 else 2560
    Kp = _rup(K, bk)
    a = jnp.pad(a, ((0, Mp - M), (0, Kp - K)))
    b = jnp.pad(b, ((0, Kp - K), (0, Np - N)))
    bias = jnp.pad(bias, (0, Np - N)).reshape(1, Np)
    nk = Kp // bk
    out = pl.pallas_call(
        functools.partial(_mm_body, nk=nk, act=act),
        grid=(Mp // bm, Np // bn, nk),
        in_specs=[
            pl.BlockSpec((bm, bk), lambda m, n, k: (m, k)),
            pl.BlockSpec((bk, bn), lambda m, n, k: (k, n)),
            pl.BlockSpec((1, bn), lambda m, n, k: (0, n)),
        ],
        out_specs=pl.BlockSpec((bm, bn), lambda m, n, k: (m, n)),
        out_shape=jax.ShapeDtypeStruct((Mp, Np), jnp.float32),
        scratch_shapes=[pltpu.VMEM((bm, bn), jnp.float32)],
        compiler_params=pltpu.CompilerParams(
            dimension_semantics=("parallel", "parallel", "arbitrary")),
    )(a, b, bias)
    return out[:M, :N]


def _edge_y_body(xls_ref, xrd_ref, attr_ref, wc_ref, bc_ref, att_ref,
                 y_ref, ex_ref, *, bm):
    xls = xls_ref[...]
    pe = jnp.dot(attr_ref[...], wc_ref[...],
                 preferred_element_type=jnp.float32) + bc_ref[...]
    s = xls + xrd_ref[...] + pe
    m = jnp.where(s > 0, s, 0.2 * s)
    ex = jnp.exp(jnp.sum(m * att_ref[...], axis=1, keepdims=True))
    row = (pl.program_id(0) * bm
           + lax.broadcasted_iota(jnp.int32, (bm, 1), 0))
    ex = jnp.where(row < _E, ex, 0.0)
    y_ref[...] = ex * xls
    ex_ref[...] = ex


@jax.jit
def _edge_y(xlsrc, xrdst, attr_pad, Wc, bc, att):
    E, H = xlsrc.shape
    bm = 1280
    return pl.pallas_call(
        functools.partial(_edge_y_body, bm=bm),
        grid=(E // bm,),
        in_specs=[
            pl.BlockSpec((bm, H), lambda m: (m, 0)),
            pl.BlockSpec((bm, H), lambda m: (m, 0)),
            pl.BlockSpec((bm, 16), lambda m: (m, 0)),
            pl.BlockSpec((16, H), lambda m: (0, 0)),
            pl.BlockSpec((1, H), lambda m: (0, 0)),
            pl.BlockSpec((1, H), lambda m: (0, 0)),
        ],
        out_specs=[
            pl.BlockSpec((bm, _HE), lambda m: (m, 0)),
            pl.BlockSpec((bm, 1), lambda m: (m, 0)),
        ],
        out_shape=[
            jax.ShapeDtypeStruct((E, _HE), jnp.float32),
            jax.ShapeDtypeStruct((E, 1), jnp.float32),
        ],
        compiler_params=pltpu.CompilerParams(
            dimension_semantics=("parallel",)),
    )(xlsrc, xrdst, attr_pad, Wc, bc.reshape(1, H), att.reshape(1, H))


def _combine_body(xl_ref, xr_ref, pel_ref, att_ref, p0_ref, p1_ref,
                  den_ref, bias_ref, pre_ref, o_ref):
    xl = xl_ref[...]
    s = xl + xr_ref[...] + pel_ref[...]
    m = jnp.where(s > 0, s, 0.2 * s)
    exl = jnp.exp(jnp.sum(m * att_ref[...], axis=1, keepdims=True))
    num = p0_ref[0] + p1_ref[0]
    r = (num + exl * xl) / (den_ref[...] + exl + 1e-16)
    r = r + bias_ref[...]
    o_ref[...] = _gelu(r) + pre_ref[...]


@jax.jit
def _gat_combine(xl, xr, pe_loop, att, parts, den, bias, pre):
    N, H = xl.shape
    bm = 1000
    return pl.pallas_call(
        _combine_body,
        grid=(N // bm,),
        in_specs=[
            pl.BlockSpec((bm, H), lambda m: (m, 0)),
            pl.BlockSpec((bm, H), lambda m: (m, 0)),
            pl.BlockSpec((bm, H), lambda m: (m, 0)),
            pl.BlockSpec((1, H), lambda m: (0, 0)),
            pl.BlockSpec((1, bm, _HE), lambda m: (0, m, 0)),
            pl.BlockSpec((1, bm, _HE), lambda m: (1, m, 0)),
            pl.BlockSpec((bm, 1), lambda m: (m, 0)),
            pl.BlockSpec((1, H), lambda m: (0, 0)),
            pl.BlockSpec((bm, H), lambda m: (m, 0)),
        ],
        out_specs=pl.BlockSpec((bm, H), lambda m: (m, 0)),
        out_shape=jax.ShapeDtypeStruct((N, H), jnp.float32),
        compiler_params=pltpu.CompilerParams(
            dimension_semantics=("parallel",)),
    )(xl, xr, pe_loop, att.reshape(1, H), parts, parts,
      den.reshape(N, 1), bias.reshape(1, H), pre)


def _gat_pair(x1, p1, x2, p2, src_pad, dst_pad, dst_bin, attr_pad, am16,
              lin_W, lin_b, pre1, pre2, zeros_acc):
    mms = []
    for p, x in ((p1, x1), (p2, x2)):
        Wc = lin_W @ p['We']
        bc = lin_b @ p['We']
        mms.append((_mm(x, p['Wl'], p['bl']), _mm(x, p['Wr'], p['br']),
                    Wc, bc, _mm(am16, Wc, bc)))
    (xl1, xr1, Wc1, bc1, pel1), (xl2, xr2, Wc2, bc2, pel2) = mms
    xls1, xrd1, xls2, xrd2 = _sc_gather4(xl1, xr1, xl2, xr2,
                                         src_pad, dst_pad)
    y1, ex1 = _edge_y(xls1, xrd1, attr_pad, Wc1, bc1, p1['att'])
    y2, ex2 = _edge_y(xls2, xrd2, attr_pad, Wc2, bc2, p2['att'])
    parts1 = _sc_scatter(y1, dst_bin, zeros_acc)
    parts2 = _sc_scatter(y2, dst_bin, zeros_acc)
    den1 = jax.ops.segment_sum(ex1[:_E, 0], dst_pad[:_E],
                               num_segments=NUM_NODES)
    den2 = jax.ops.segment_sum(ex2[:_E, 0], dst_pad[:_E],
                               num_segments=NUM_NODES)
    o1 = _gat_combine(xl1, xr1, pel1, p1['att'], parts1, den1,
                      p1['bias'], pre1)
    o2 = _gat_combine(xl2, xr2, pel2, p2['att'], parts2, den2,
                      p2['bias'], pre2)
    return o1, o2


def _gat_seg(xf, mm, p):
    N = xf.shape[0]
    mask = mm != 0.0
    cnt = jnp.sum(mask.astype(jnp.float32), axis=0)
    loop_attr = jnp.sum(mm, axis=0) / jnp.maximum(cnt, 1.0)
    xl = _mm(xf, p['Wl'], p['bl'])
    xr = _mm(xf, p['Wr'], p['br'])
    We = p['We'][0]
    s = xl[:, None, :] + xr[None, :, :] + mm[:, :, None] * We[None, None, :]
    m = jnp.where(s > 0, s, 0.2 * s)
    alpha = jnp.einsum('ijh,h->ij', m, p['att'])
    alpha = jnp.where(mask, alpha, -jnp.inf)
    sl = xl + xr + loop_attr[:, None] * We[None, :]
    ml = jnp.where(sl > 0, sl, 0.2 * sl)
    aloop = ml @ p['att']
    amax = jnp.maximum(jnp.max(jnp.where(mask, alpha, -jnp.inf), axis=0), aloop)
    ex = jnp.where(mask, jnp.exp(alpha - amax[None, :]), 0.0)
    exl = jnp.exp(aloop - amax)
    den = jnp.sum(ex, axis=0) + exl + 1e-16
    out = (ex.T @ xl + exl[:, None] * xl) / den[:, None]
    return out + p['bias']


def kernel(x, attr, segment_edge, segment_node, params, s_index, e_index,
           edge_index, cur_t, cur_w):
    p = params
    mm = segment_node @ segment_node.T
    src, dst = edge_index[0], edge_index[1]

    mask_idx = (jnp.sum(x, axis=1, keepdims=True) != XMIN * 4).astype(jnp.float32)
    ratio = 0.8 + 0.4 * jax.random.uniform(jax.random.key(1))
    drop_idx = (jax.random.uniform(jax.random.key(2), (x.shape[0], 1)) > 0.3
                ).astype(jnp.float32)
    x_norm = ((x - XMIN) / (XMAX - XMIN) * ratio * drop_idx).T
    h = _mm(x_norm, p['fc1_W'], p['fc1_b'], act='gelu')
    mu = _mm(h, p['fc2_W'], p['fc2_b'])
    log_var = _mm(h, p['fc3_W'], p['fc3_b'])
    eps = jax.random.normal(jax.random.key(3), mu.shape, jnp.float32)
    z = mu + eps * jnp.exp(log_var / 2)
    hz = _mm(z, p['fc4_W'], p['fc4_b'], act='gelu')
    x_rec = _mm(hz, p['fc5_W'], p['fc5_b'])
    x_rec = (x_rec / ratio).T
    x_rec = x_rec * (XMAX - XMIN) + XMIN
    x_rec1 = mask_idx * x + (1 - mask_idx) * x_rec

    ones = jnp.ones(dst.shape[0], jnp.float32)
    cnt = jax.ops.segment_sum(ones, dst, num_segments=NUM_NODES)
    am16 = (jax.ops.segment_sum(attr, dst, num_segments=NUM_NODES)
            / jnp.maximum(cnt, 1.0)[:, None])

    npad = _EP - _E
    src_pad = jnp.pad(src, (0, npad)).astype(jnp.int32)
    dst_pad = jnp.pad(dst, (0, npad)).astype(jnp.int32)
    dst_bin = jnp.pad(dst, (0, npad),
                      constant_values=NUM_NODES).astype(jnp.int32)
    attr_pad = jnp.pad(attr, ((0, npad), (0, 0)))
    zeros_acc = jnp.zeros((_NBIN, _HE), jnp.float32)

    ne = p['node_embed']
    pre1 = ne
    data = _mm(x_rec1, p['node_lin_W'], p['node_lin_b'], act='gelu')
    pre2 = data
    for layer in ('0', '1'):
        ne, data = _gat_pair(ne, p['conv1_' + layer], data,
                             p['conv2_' + layer], src_pad, dst_pad, dst_bin,
                             attr_pad, am16, p['attr_lin_W'],
                             p['attr_lin_b'], pre1, pre2, zeros_acc)

    time_embed = p['time_embed'][cur_t]
    week_embed = p['week_embed'][cur_w]
    seg_embed = p['segment_embed'][s_index]
    h1 = _mm(attr, p['attr1_W1'], p['attr1_b1'], act='leaky01')
    attr2 = (_mm(_mm(segment_edge, h1), p['attr1_W2'])
             + jnp.sum(segment_edge, axis=1, keepdims=True)
             * p['attr1_b2'][None, :])
    embed = jnp.take(p['edge_embed'], e_index, axis=0)
    attr3 = _mm(segment_edge, embed)
    xnh = _mm(segment_node, jnp.concatenate([data, ne], axis=1))
    x2, x1 = xnh[:, :128], xnh[:, 128:]
    xf = jnp.concatenate([seg_embed, time_embed, attr2, attr3, x2, x1,
                          week_embed], axis=1)
    xf1 = _gelu(_gat_seg(xf, mm, p['conv3']))
    for name in ('lin0', 'lin1', 'lin2'):
        xf = _mm(xf, p[name + '_W'], p[name + '_b'], act='gelu')
    xf = _mm(jnp.concatenate([xf, xf1], axis=1), p['lin3_W'], p['lin3_b'])
    out = jax.nn.sigmoid(xf) * 3600.0
    return out, x_rec
